# Initial kernel scaffold; baseline (speedup 1.0000x reference)
import functools

import jax
import jax.numpy as jnp
from jax import lax
from jax.experimental import pallas as pl
from jax.experimental.pallas import tpu as pltpu
from jax.experimental.pallas import tpu_sc as plsc

NC = 2
NS = 16
NW = NC * NS
CHUNK = 128


def _sc_mesh():
    return plsc.VectorSubcoreMesh(core_axis_name="c", subcore_axis_name="s")


def _make_deg_kernel(A, n_chunks):
    @functools.partial(
        pl.kernel,
        out_type=jax.ShapeDtypeStruct((NC, A), jnp.float32),
        mesh=_sc_mesh(),
        scratch_types=[
            pltpu.VMEM((CHUNK,), jnp.int32),
            pltpu.VMEM((CHUNK,), jnp.float32),
            pltpu.VMEM((A // NS,), jnp.float32),
            pltpu.VMEM_SHARED((A,), jnp.float32),
        ],
    )
    def deg_kernel(dst_hbm, out_hbm, didx, ones, vbuf, dacc):
        c = lax.axis_index("c")
        s = lax.axis_index("s")
        w = c * NS + s
        per_tile = A // NS
        zeros16 = jnp.zeros((16,), jnp.float32)
        ones16 = jnp.ones((16,), jnp.float32)

        def zb(i, _):
            vbuf[pl.ds(i * 16, 16)] = zeros16
            return 0

        lax.fori_loop(0, per_tile // 16, zb, 0)
        for j in range(CHUNK // 16):
            ones[pl.ds(j * 16, 16)] = ones16
        pltpu.sync_copy(vbuf, dacc.at[pl.ds(s * per_tile, per_tile)])
        plsc.subcore_barrier()

        def body(i, _):
            pltpu.sync_copy(dst_hbm.at[w, i], didx)
            pltpu.sync_copy(ones, dacc.at[didx], add=True)
            return 0

        lax.fori_loop(0, n_chunks, body, 0)
        plsc.subcore_barrier()
        pltpu.sync_copy(dacc.at[pl.ds(s * per_tile, per_tile)], vbuf)
        pltpu.sync_copy(vbuf, out_hbm.at[c, pl.ds(s * per_tile, per_tile)])

    return deg_kernel


def _make_scatter_kernel(A, D, n_chunks):
    @functools.partial(
        pl.kernel,
        out_type=jax.ShapeDtypeStruct((NC, A, D), jnp.float32),
        mesh=_sc_mesh(),
        scratch_types=[
            pltpu.VMEM((CHUNK,), jnp.int32),
            pltpu.VMEM((CHUNK,), jnp.int32),
            pltpu.VMEM((CHUNK, D), jnp.float32),
            pltpu.VMEM_SHARED((A, D), jnp.float32),
            pltpu.SemaphoreType.DMA,
        ],
    )
    def scatter_kernel(h_hbm, src_hbm, dst_hbm, out_hbm, sidx, didx, rows,
                       acc, sem):
        c = lax.axis_index("c")
        s = lax.axis_index("s")
        w = c * NS + s
        per_tile = A // NS
        zeros16 = jnp.zeros((16,), jnp.float32)

        def zb(i, _):
            r = i // (D // 16)
            col = (i % (D // 16)) * 16
            rows[r, pl.ds(col, 16)] = zeros16
            return 0

        lax.fori_loop(0, CHUNK * (D // 16), zb, 0)
        for k in range(per_tile // CHUNK):
            pltpu.sync_copy(rows, acc.at[pl.ds(s * per_tile + k * CHUNK, CHUNK)])
        plsc.subcore_barrier()

        def body(i, _):
            pltpu.sync_copy(src_hbm.at[w, i], sidx)
            pltpu.async_copy(h_hbm.at[sidx], rows, sem).wait()
            pltpu.sync_copy(dst_hbm.at[w, i], didx)
            pltpu.sync_copy(rows, acc.at[didx], add=True)
            return 0

        lax.fori_loop(0, n_chunks, body, 0)
        plsc.subcore_barrier()
        for k in range(per_tile // CHUNK):
            row0 = s * per_tile + k * CHUNK
            pltpu.sync_copy(acc.at[pl.ds(row0, CHUNK)], rows)
            pltpu.sync_copy(rows, out_hbm.at[c, pl.ds(row0, CHUNK)])

    return scatter_kernel


def _pre_body(x_ref, w1_ref, wsk_ref, bsk_ref, d0_ref, d1_ref,
              hs1_ref, skip_ref, dinv_ref):
    deg = d0_ref[...] + d1_ref[...] + 1.0
    dinv = lax.rsqrt(deg)
    h1 = jnp.dot(x_ref[...], w1_ref[...], preferred_element_type=jnp.float32)
    hs1_ref[...] = h1 * dinv
    skip_ref[...] = (
        jnp.dot(x_ref[...], wsk_ref[...], preferred_element_type=jnp.float32)
        + bsk_ref[...])
    dinv_ref[...] = dinv


def _layer_norm(u, g, b):
    m = jnp.mean(u, axis=-1, keepdims=True)
    v = jnp.mean((u - m) ** 2, axis=-1, keepdims=True)
    return (u - m) * lax.rsqrt(v + 1e-5) * g + b


def _mid_body(p0_ref, p1_ref, hs1_ref, skip_ref, dinv_ref, b1_ref, g1_ref,
              be1_ref, w2_ref, hs2_ref):
    dinv = dinv_ref[...]
    agg = (p0_ref[...] + p1_ref[...] + hs1_ref[...]) * dinv + b1_ref[...]
    h = jax.nn.relu(_layer_norm(agg, g1_ref[...], be1_ref[...]))
    u = skip_ref[...] + h
    hs2_ref[...] = (
        jnp.dot(u, w2_ref[...], preferred_element_type=jnp.float32) * dinv)


def _post_body(q0_ref, q1_ref, hs2_ref, dinv_ref, b2_ref, g2_ref, be2_ref,
               lo_r_ref, hi_r_ref, lo_c_ref, hi_c_ref,
               h2_ref, pool_ref, pool_acc, *, rows_per_blk, n_blk):
    agg = ((q0_ref[...] + q1_ref[...] + hs2_ref[...]) * dinv_ref[...]
           + b2_ref[...])
    y = jax.nn.relu(_layer_norm(agg, g2_ref[...], be2_ref[...]))
    h2_ref[...] = y
    i = pl.program_id(0)
    ridx = (lax.broadcasted_iota(jnp.float32, (rows_per_blk, 1), 0)
            + jnp.float32(rows_per_blk) * i.astype(jnp.float32))
    onehot = ((ridx >= lo_r_ref[...]) & (ridx < hi_r_ref[...])
              ).astype(jnp.float32)
    part = lax.dot_general(onehot, y, (((0,), (0,)), ((), ())),
                           preferred_element_type=jnp.float32)

    @pl.when(i == 0)
    def _():
        pool_acc[...] = jnp.zeros_like(pool_acc)

    pool_acc[...] += part

    @pl.when(i == n_blk - 1)
    def _():
        cnt = jnp.maximum(hi_c_ref[...] - lo_c_ref[...], 1.0)
        pool_ref[...] = pool_acc[...] / cnt


def kernel(x, edge_index, ptr, W1, b1, g1, be1, W2, b2, g2, be2, Wsk, bsk):
    N, D = x.shape
    E = edge_index.shape[1]
    NB = ptr.shape[0] - 1

    PAD_ROWS = 240
    A = N + PAD_ROWS
    assert A % (NS * CHUNK) == 0

    src = edge_index[0].astype(jnp.int32)
    dst = edge_index[1].astype(jnp.int32)
    n_chunks = -(-E // (NW * CHUNK))
    Ep = NW * n_chunks * CHUNK
    pad = Ep - E
    pad_pos = jnp.arange(pad, dtype=jnp.int32)
    src_p = jnp.concatenate([src, (pad_pos * 97) % N])
    dst_p = jnp.concatenate([dst, N + (pad_pos % PAD_ROWS)])
    src_p = src_p.reshape(NW, n_chunks, CHUNK)
    dst_p = dst_p.reshape(NW, n_chunks, CHUNK)

    degp = _make_deg_kernel(A, n_chunks)(dst_p)
    d0 = degp[0, :N, None]
    d1 = degp[1, :N, None]

    R = 1000
    n_blk = N // R
    row_blk = pl.BlockSpec((R, D), lambda i: (i, 0))
    col1_blk = pl.BlockSpec((R, 1), lambda i: (i, 0))
    w_blk = pl.BlockSpec((D, D), lambda i: (0, 0))
    vec_blk = pl.BlockSpec((1, D), lambda i: (0, 0))

    hs1, skip, dinv = pl.pallas_call(
        _pre_body,
        grid=(n_blk,),
        in_specs=[row_blk, w_blk, w_blk, vec_blk, col1_blk, col1_blk],
        out_specs=[row_blk, row_blk, col1_blk],
        out_shape=[
            jax.ShapeDtypeStruct((N, D), jnp.float32),
            jax.ShapeDtypeStruct((N, D), jnp.float32),
            jax.ShapeDtypeStruct((N, 1), jnp.float32),
        ],
    )(x, W1, Wsk, bsk[None, :], d0, d1)

    scatter = _make_scatter_kernel(A, D, n_chunks)
    s1 = scatter(hs1, src_p, dst_p)

    hs2 = pl.pallas_call(
        _mid_body,
        grid=(n_blk,),
        in_specs=[row_blk, row_blk, row_blk, row_blk, col1_blk,
                  vec_blk, vec_blk, vec_blk, w_blk],
        out_specs=row_blk,
        out_shape=jax.ShapeDtypeStruct((N, D), jnp.float32),
    )(s1[0, :N], s1[1, :N], hs1, skip, dinv,
      b1[None, :], g1[None, :], be1[None, :], W2)

    s2 = scatter(hs2, src_p, dst_p)

    ptr_f = ptr.astype(jnp.float32)
    lo_r = ptr_f[:-1][None, :]
    hi_r = ptr_f[1:][None, :]
    lo_c = ptr_f[:-1][:, None]
    hi_c = ptr_f[1:][:, None]

    h2, pooled = pl.pallas_call(
        functools.partial(_post_body, rows_per_blk=R, n_blk=n_blk),
        grid=(n_blk,),
        in_specs=[row_blk, row_blk, row_blk, col1_blk,
                  vec_blk, vec_blk, vec_blk,
                  pl.BlockSpec((1, NB), lambda i: (0, 0)),
                  pl.BlockSpec((1, NB), lambda i: (0, 0)),
                  pl.BlockSpec((NB, 1), lambda i: (0, 0)),
                  pl.BlockSpec((NB, 1), lambda i: (0, 0))],
        out_specs=[row_blk, pl.BlockSpec((NB, D), lambda i: (0, 0))],
        out_shape=[
            jax.ShapeDtypeStruct((N, D), jnp.float32),
            jax.ShapeDtypeStruct((NB, D), jnp.float32),
        ],
        scratch_shapes=[pltpu.VMEM((NB, D), jnp.float32)],
    )(s2[0, :N], s2[1, :N], hs2, dinv,
      b2[None, :], g2[None, :], be2[None, :], lo_r, hi_r, lo_c, hi_c)

    return (h2, pooled)


# speedup vs baseline: 15.7594x; 15.7594x over previous
import functools

import jax
import jax.numpy as jnp
from jax import lax
from jax.experimental import pallas as pl
from jax.experimental.pallas import tpu as pltpu
from jax.experimental.pallas import tpu_sc as plsc

NC = 2
NS = 16
NW = NC * NS
CHUNK = 128


def _sc_mesh():
    return plsc.VectorSubcoreMesh(core_axis_name="c", subcore_axis_name="s",
                                  num_cores=NC, num_subcores=NS)


def _make_deg_kernel(A, n_chunks):
    @functools.partial(
        pl.kernel,
        out_type=jax.ShapeDtypeStruct((NC, A), jnp.float32),
        mesh=_sc_mesh(),
        scratch_types=[
            pltpu.VMEM((CHUNK,), jnp.int32),
            pltpu.VMEM((CHUNK,), jnp.float32),
            pltpu.VMEM((A // NS,), jnp.float32),
            pltpu.VMEM_SHARED((A,), jnp.float32),
        ],
    )
    def deg_kernel(dst_hbm, out_hbm, didx, ones, vbuf, dacc):
        c = lax.axis_index("c")
        s = lax.axis_index("s")
        w = c * NS + s
        per_tile = A // NS
        zeros16 = jnp.zeros((16,), jnp.float32)
        ones16 = jnp.ones((16,), jnp.float32)

        def zb(i, _):
            vbuf[pl.ds(i * 16, 16)] = zeros16
            return 0

        lax.fori_loop(0, per_tile // 16, zb, 0)
        for j in range(CHUNK // 16):
            ones[pl.ds(j * 16, 16)] = ones16
        pltpu.sync_copy(vbuf, dacc.at[pl.ds(s * per_tile, per_tile)])
        plsc.subcore_barrier()

        def body(i, _):
            pltpu.sync_copy(dst_hbm.at[w, i], didx)
            pltpu.sync_copy(ones, dacc.at[didx], add=True)
            return 0

        lax.fori_loop(0, n_chunks, body, 0)
        plsc.subcore_barrier()
        pltpu.sync_copy(dacc.at[pl.ds(s * per_tile, per_tile)], vbuf)
        pltpu.sync_copy(vbuf, out_hbm.at[c, pl.ds(s * per_tile, per_tile)])

    return deg_kernel


def _make_scatter_kernel(A, D, n_chunks):
    @functools.partial(
        pl.kernel,
        out_type=jax.ShapeDtypeStruct((NC, A, D), jnp.float32),
        mesh=_sc_mesh(),
        scratch_types=[
            pltpu.VMEM((CHUNK,), jnp.int32),
            pltpu.VMEM((CHUNK,), jnp.int32),
            pltpu.VMEM((CHUNK, D), jnp.float32),
            pltpu.VMEM_SHARED((A, D), jnp.float32),
            pltpu.SemaphoreType.DMA,
        ],
    )
    def scatter_kernel(h_hbm, src_hbm, dst_hbm, out_hbm, sidx, didx, rows,
                       acc, sem):
        c = lax.axis_index("c")
        s = lax.axis_index("s")
        w = c * NS + s
        per_tile = A // NS
        zeros16 = jnp.zeros((16,), jnp.float32)

        def zb(i, _):
            r = i // (D // 16)
            col = (i % (D // 16)) * 16
            rows[r, pl.ds(col, 16)] = zeros16
            return 0

        lax.fori_loop(0, CHUNK * (D // 16), zb, 0)
        for k in range(per_tile // CHUNK):
            pltpu.sync_copy(rows, acc.at[pl.ds(s * per_tile + k * CHUNK, CHUNK)])
        plsc.subcore_barrier()

        def body(i, _):
            pltpu.sync_copy(src_hbm.at[w, i], sidx)
            pltpu.async_copy(h_hbm.at[sidx], rows, sem).wait()
            pltpu.sync_copy(dst_hbm.at[w, i], didx)
            pltpu.sync_copy(rows, acc.at[didx], add=True)
            return 0

        lax.fori_loop(0, n_chunks, body, 0)
        plsc.subcore_barrier()
        for k in range(per_tile // CHUNK):
            row0 = s * per_tile + k * CHUNK
            pltpu.sync_copy(acc.at[pl.ds(row0, CHUNK)], rows)
            pltpu.sync_copy(rows, out_hbm.at[c, pl.ds(row0, CHUNK)])

    return scatter_kernel


def _pre_body(x_ref, w1_ref, wsk_ref, bsk_ref, d0_ref, d1_ref,
              hs1_ref, skip_ref, dinv_ref):
    deg = d0_ref[...] + d1_ref[...] + 1.0
    dinv = lax.rsqrt(deg)
    h1 = jnp.dot(x_ref[...], w1_ref[...], preferred_element_type=jnp.float32)
    hs1_ref[...] = h1 * dinv
    skip_ref[...] = (
        jnp.dot(x_ref[...], wsk_ref[...], preferred_element_type=jnp.float32)
        + bsk_ref[...])
    dinv_ref[...] = dinv


def _layer_norm(u, g, b):
    m = jnp.mean(u, axis=-1, keepdims=True)
    v = jnp.mean((u - m) ** 2, axis=-1, keepdims=True)
    return (u - m) * lax.rsqrt(v + 1e-5) * g + b


def _mid_body(p0_ref, p1_ref, hs1_ref, skip_ref, dinv_ref, b1_ref, g1_ref,
              be1_ref, w2_ref, hs2_ref):
    dinv = dinv_ref[...]
    agg = (p0_ref[...] + p1_ref[...] + hs1_ref[...]) * dinv + b1_ref[...]
    h = jax.nn.relu(_layer_norm(agg, g1_ref[...], be1_ref[...]))
    u = skip_ref[...] + h
    hs2_ref[...] = (
        jnp.dot(u, w2_ref[...], preferred_element_type=jnp.float32) * dinv)


def _post_body(q0_ref, q1_ref, hs2_ref, dinv_ref, b2_ref, g2_ref, be2_ref,
               lo_r_ref, hi_r_ref, lo_c_ref, hi_c_ref,
               h2_ref, pool_ref, pool_acc, *, rows_per_blk, n_blk):
    agg = ((q0_ref[...] + q1_ref[...] + hs2_ref[...]) * dinv_ref[...]
           + b2_ref[...])
    y = jax.nn.relu(_layer_norm(agg, g2_ref[...], be2_ref[...]))
    h2_ref[...] = y
    i = pl.program_id(0)
    ridx = (lax.broadcasted_iota(jnp.int32, (rows_per_blk, 1), 0)
            + rows_per_blk * i).astype(jnp.float32)
    onehot = ((ridx >= lo_r_ref[...]) & (ridx < hi_r_ref[...])
              ).astype(jnp.float32)
    part = lax.dot_general(onehot, y, (((0,), (0,)), ((), ())),
                           preferred_element_type=jnp.float32)

    @pl.when(i == 0)
    def _():
        pool_acc[...] = jnp.zeros_like(pool_acc)

    pool_acc[...] += part

    @pl.when(i == n_blk - 1)
    def _():
        cnt = jnp.maximum(hi_c_ref[...] - lo_c_ref[...], 1.0)
        pool_ref[...] = pool_acc[...] / cnt


def kernel(x, edge_index, ptr, W1, b1, g1, be1, W2, b2, g2, be2, Wsk, bsk):
    N, D = x.shape
    E = edge_index.shape[1]
    NB = ptr.shape[0] - 1

    PAD_ROWS = 240
    A = N + PAD_ROWS
    assert A % (NS * CHUNK) == 0

    src = edge_index[0].astype(jnp.int32)
    dst = edge_index[1].astype(jnp.int32)
    n_chunks = -(-E // (NW * CHUNK))
    Ep = NW * n_chunks * CHUNK
    pad = Ep - E
    pad_pos = jnp.arange(pad, dtype=jnp.int32)
    src_p = jnp.concatenate([src, (pad_pos * 97) % N])
    dst_p = jnp.concatenate([dst, N + (pad_pos % PAD_ROWS)])
    src_p = src_p.reshape(NW, n_chunks, CHUNK)
    dst_p = dst_p.reshape(NW, n_chunks, CHUNK)

    degp = _make_deg_kernel(A, n_chunks)(dst_p)
    d0 = degp[0, :N, None]
    d1 = degp[1, :N, None]

    R = 1000
    n_blk = N // R
    row_blk = pl.BlockSpec((R, D), lambda i: (i, 0))
    col1_blk = pl.BlockSpec((R, 1), lambda i: (i, 0))
    w_blk = pl.BlockSpec((D, D), lambda i: (0, 0))
    vec_blk = pl.BlockSpec((1, D), lambda i: (0, 0))

    hs1, skip, dinv = pl.pallas_call(
        _pre_body,
        grid=(n_blk,),
        in_specs=[row_blk, w_blk, w_blk, vec_blk, col1_blk, col1_blk],
        out_specs=[row_blk, row_blk, col1_blk],
        out_shape=[
            jax.ShapeDtypeStruct((N, D), jnp.float32),
            jax.ShapeDtypeStruct((N, D), jnp.float32),
            jax.ShapeDtypeStruct((N, 1), jnp.float32),
        ],
    )(x, W1, Wsk, bsk[None, :], d0, d1)

    scatter = _make_scatter_kernel(A, D, n_chunks)
    s1 = scatter(hs1, src_p, dst_p)

    hs2 = pl.pallas_call(
        _mid_body,
        grid=(n_blk,),
        in_specs=[row_blk, row_blk, row_blk, row_blk, col1_blk,
                  vec_blk, vec_blk, vec_blk, w_blk],
        out_specs=row_blk,
        out_shape=jax.ShapeDtypeStruct((N, D), jnp.float32),
    )(s1[0, :N], s1[1, :N], hs1, skip, dinv,
      b1[None, :], g1[None, :], be1[None, :], W2)

    s2 = scatter(hs2, src_p, dst_p)

    ptr_f = ptr.astype(jnp.float32)
    lo_r = ptr_f[:-1][None, :]
    hi_r = ptr_f[1:][None, :]
    lo_c = ptr_f[:-1][:, None]
    hi_c = ptr_f[1:][:, None]

    h2, pooled = pl.pallas_call(
        functools.partial(_post_body, rows_per_blk=R, n_blk=n_blk),
        grid=(n_blk,),
        in_specs=[row_blk, row_blk, row_blk, col1_blk,
                  vec_blk, vec_blk, vec_blk,
                  pl.BlockSpec((1, NB), lambda i: (0, 0)),
                  pl.BlockSpec((1, NB), lambda i: (0, 0)),
                  pl.BlockSpec((NB, 1), lambda i: (0, 0)),
                  pl.BlockSpec((NB, 1), lambda i: (0, 0))],
        out_specs=[row_blk, pl.BlockSpec((NB, D), lambda i: (0, 0))],
        out_shape=[
            jax.ShapeDtypeStruct((N, D), jnp.float32),
            jax.ShapeDtypeStruct((NB, D), jnp.float32),
        ],
        scratch_shapes=[pltpu.VMEM((NB, D), jnp.float32)],
    )(s2[0, :N], s2[1, :N], hs2, dinv,
      b2[None, :], g2[None, :], be2[None, :], lo_r, hi_r, lo_c, hi_c)

    return (h2, pooled)


# speedup vs baseline: 19.7194x; 1.2513x over previous
import functools

import jax
import jax.numpy as jnp
from jax import lax
from jax.experimental import pallas as pl
from jax.experimental.pallas import tpu as pltpu
from jax.experimental.pallas import tpu_sc as plsc

NC = 2
NS = 16
NW = NC * NS
CHUNK = 128


def _sc_mesh():
    return plsc.VectorSubcoreMesh(core_axis_name="c", subcore_axis_name="s",
                                  num_cores=NC, num_subcores=NS)


def _make_deg_kernel(A, n_chunks):
    @functools.partial(
        pl.kernel,
        out_type=jax.ShapeDtypeStruct((NC, A), jnp.float32),
        mesh=_sc_mesh(),
        scratch_types=[
            pltpu.VMEM((n_chunks + 1, CHUNK), jnp.int32),
            pltpu.VMEM((CHUNK,), jnp.float32),
            pltpu.VMEM((A // NS,), jnp.float32),
            pltpu.VMEM_SHARED((A,), jnp.float32),
            pltpu.SemaphoreType.DMA,
            pltpu.SemaphoreType.DMA,
        ],
    )
    def deg_kernel(dst_hbm, out_hbm, didx, ones, vbuf, dacc, sem0, sem1):
        c = lax.axis_index("c")
        s = lax.axis_index("s")
        w = c * NS + s
        per_tile = A // NS
        zeros16 = jnp.zeros((16,), jnp.float32)
        ones16 = jnp.ones((16,), jnp.float32)
        sems = (sem0, sem1)

        pltpu.sync_copy(dst_hbm.at[w], didx)

        def zb(i, _):
            vbuf[pl.ds(i * 16, 16)] = zeros16
            return 0

        lax.fori_loop(0, per_tile // 16, zb, 0)
        for j in range(CHUNK // 16):
            ones[pl.ds(j * 16, 16)] = ones16
        pltpu.sync_copy(vbuf, dacc.at[pl.ds(s * per_tile, per_tile)])
        plsc.subcore_barrier()

        def pair(p, _):
            for b in range(2):
                i = 2 * p + b
                o = 1 - b
                pltpu.async_copy(ones, dacc.at[didx.at[i]], sems[b], add=True)

                @pl.when(i > 0)
                def _():
                    pltpu.make_async_copy(
                        ones, dacc.at[didx.at[i - 1]], sems[o]).wait()
            return 0

        lax.fori_loop(0, n_chunks // 2, pair, 0)
        pltpu.make_async_copy(
            ones, dacc.at[didx.at[n_chunks - 1]], sems[1]).wait()
        plsc.subcore_barrier()
        pltpu.sync_copy(dacc.at[pl.ds(s * per_tile, per_tile)], vbuf)
        pltpu.sync_copy(vbuf, out_hbm.at[c, pl.ds(s * per_tile, per_tile)])

    return deg_kernel


def _make_scatter_kernel(A, D, n_chunks):
    @functools.partial(
        pl.kernel,
        out_type=jax.ShapeDtypeStruct((NC, A, D), jnp.float32),
        mesh=_sc_mesh(),
        scratch_types=[
            pltpu.VMEM((CHUNK,), jnp.int32),
            pltpu.VMEM((CHUNK,), jnp.int32),
            pltpu.VMEM((CHUNK,), jnp.int32),
            pltpu.VMEM((CHUNK,), jnp.int32),
            pltpu.VMEM((CHUNK, D), jnp.float32),
            pltpu.VMEM((CHUNK, D), jnp.float32),
            pltpu.VMEM_SHARED((A, D), jnp.float32),
            pltpu.SemaphoreType.DMA,
            pltpu.SemaphoreType.DMA,
            pltpu.SemaphoreType.DMA,
            pltpu.SemaphoreType.DMA,
        ],
    )
    def scatter_kernel(h_hbm, src_hbm, dst_hbm, out_hbm, sidx0, sidx1,
                       didx0, didx1, rows0, rows1, acc,
                       gsem0, gsem1, ssem0, ssem1):
        c = lax.axis_index("c")
        s = lax.axis_index("s")
        w = c * NS + s
        per_tile = A // NS
        zeros16 = jnp.zeros((16,), jnp.float32)
        sidx = (sidx0, sidx1)
        didx = (didx0, didx1)
        rows = (rows0, rows1)
        gsem = (gsem0, gsem1)
        ssem = (ssem0, ssem1)

        def zb(i, _):
            r = i // (D // 16)
            col = (i % (D // 16)) * 16
            rows0[r, pl.ds(col, 16)] = zeros16
            return 0

        lax.fori_loop(0, CHUNK * (D // 16), zb, 0)
        for k in range(per_tile // CHUNK):
            pltpu.sync_copy(rows0, acc.at[pl.ds(s * per_tile + k * CHUNK, CHUNK)])
        plsc.subcore_barrier()

        pltpu.sync_copy(src_hbm.at[w, 0], sidx0)
        pltpu.sync_copy(dst_hbm.at[w, 0], didx0)
        pltpu.async_copy(h_hbm.at[sidx0], rows0, gsem0)

        def pair(p, _):
            for b in range(2):
                i = 2 * p + b
                o = 1 - b
                pltpu.make_async_copy(h_hbm.at[sidx[b]], rows[b],
                                      gsem[b]).wait()
                pltpu.async_copy(rows[b], acc.at[didx[b]], ssem[b],
                                 add=True)

                @pl.when(i > 0)
                def _():
                    pltpu.make_async_copy(rows[o], acc.at[didx[o]],
                                          ssem[o]).wait()

                pltpu.sync_copy(src_hbm.at[w, i + 1], sidx[o])
                pltpu.sync_copy(dst_hbm.at[w, i + 1], didx[o])
                pltpu.async_copy(h_hbm.at[sidx[o]], rows[o], gsem[o])
            return 0

        lax.fori_loop(0, n_chunks // 2, pair, 0)
        pltpu.make_async_copy(rows1, acc.at[didx1], ssem1).wait()
        pltpu.make_async_copy(h_hbm.at[sidx0], rows0, gsem0).wait()
        plsc.subcore_barrier()
        for k in range(per_tile // CHUNK):
            row0 = s * per_tile + k * CHUNK
            pltpu.sync_copy(acc.at[pl.ds(row0, CHUNK)], rows0)
            pltpu.sync_copy(rows0, out_hbm.at[c, pl.ds(row0, CHUNK)])

    return scatter_kernel


def _pre_body(x_ref, w1_ref, wsk_ref, bsk_ref, d0_ref, d1_ref,
              hs1_ref, skip_ref, dinv_ref):
    deg = d0_ref[...] + d1_ref[...] + 1.0
    dinv = lax.rsqrt(deg)
    h1 = jnp.dot(x_ref[...], w1_ref[...], preferred_element_type=jnp.float32)
    hs1_ref[...] = h1 * dinv
    skip_ref[...] = (
        jnp.dot(x_ref[...], wsk_ref[...], preferred_element_type=jnp.float32)
        + bsk_ref[...])
    dinv_ref[...] = dinv


def _layer_norm(u, g, b):
    m = jnp.mean(u, axis=-1, keepdims=True)
    v = jnp.mean((u - m) ** 2, axis=-1, keepdims=True)
    return (u - m) * lax.rsqrt(v + 1e-5) * g + b


def _mid_body(p0_ref, p1_ref, hs1_ref, skip_ref, dinv_ref, b1_ref, g1_ref,
              be1_ref, w2_ref, hs2_ref):
    dinv = dinv_ref[...]
    agg = (p0_ref[...] + p1_ref[...] + hs1_ref[...]) * dinv + b1_ref[...]
    h = jax.nn.relu(_layer_norm(agg, g1_ref[...], be1_ref[...]))
    u = skip_ref[...] + h
    hs2_ref[...] = (
        jnp.dot(u, w2_ref[...], preferred_element_type=jnp.float32) * dinv)


def _post_body(q0_ref, q1_ref, hs2_ref, dinv_ref, b2_ref, g2_ref, be2_ref,
               lo_r_ref, hi_r_ref, lo_c_ref, hi_c_ref,
               h2_ref, pool_ref, pool_acc, *, rows_per_blk, n_blk):
    agg = ((q0_ref[...] + q1_ref[...] + hs2_ref[...]) * dinv_ref[...]
           + b2_ref[...])
    y = jax.nn.relu(_layer_norm(agg, g2_ref[...], be2_ref[...]))
    h2_ref[...] = y
    i = pl.program_id(0)
    ridx = (lax.broadcasted_iota(jnp.int32, (rows_per_blk, 1), 0)
            + rows_per_blk * i).astype(jnp.float32)
    onehot = ((ridx >= lo_r_ref[...]) & (ridx < hi_r_ref[...])
              ).astype(jnp.float32)
    part = lax.dot_general(onehot, y, (((0,), (0,)), ((), ())),
                           preferred_element_type=jnp.float32)

    @pl.when(i == 0)
    def _():
        pool_acc[...] = jnp.zeros_like(pool_acc)

    pool_acc[...] += part

    @pl.when(i == n_blk - 1)
    def _():
        cnt = jnp.maximum(hi_c_ref[...] - lo_c_ref[...], 1.0)
        pool_ref[...] = pool_acc[...] / cnt


def kernel(x, edge_index, ptr, W1, b1, g1, be1, W2, b2, g2, be2, Wsk, bsk):
    N, D = x.shape
    E = edge_index.shape[1]
    NB = ptr.shape[0] - 1

    PAD_ROWS = 240
    A = N + PAD_ROWS
    assert A % (NS * CHUNK) == 0

    src = edge_index[0].astype(jnp.int32)
    dst = edge_index[1].astype(jnp.int32)
    n_chunks = 2 * (-(-E // (NW * CHUNK * 2)))
    Ep = NW * n_chunks * CHUNK
    pad = Ep - E
    pad_pos = jnp.arange(pad, dtype=jnp.int32)
    src_p = jnp.concatenate([src, (pad_pos * 97) % N]).reshape(
        NW, n_chunks, CHUNK)
    dst_p = jnp.concatenate([dst, N + (pad_pos % PAD_ROWS)]).reshape(
        NW, n_chunks, CHUNK)
    xpos = jnp.arange(NW * CHUNK, dtype=jnp.int32)
    src_x = ((xpos * 131) % N).reshape(NW, 1, CHUNK)
    dst_x = (N + (xpos % PAD_ROWS)).reshape(NW, 1, CHUNK)
    src_p = jnp.concatenate([src_p, src_x], axis=1)
    dst_p = jnp.concatenate([dst_p, dst_x], axis=1)

    degp = _make_deg_kernel(A, n_chunks)(dst_p)
    d0 = degp[0, :N, None]
    d1 = degp[1, :N, None]

    R = 1000
    n_blk = N // R
    row_blk = pl.BlockSpec((R, D), lambda i: (i, 0))
    col1_blk = pl.BlockSpec((R, 1), lambda i: (i, 0))
    w_blk = pl.BlockSpec((D, D), lambda i: (0, 0))
    vec_blk = pl.BlockSpec((1, D), lambda i: (0, 0))

    hs1, skip, dinv = pl.pallas_call(
        _pre_body,
        grid=(n_blk,),
        in_specs=[row_blk, w_blk, w_blk, vec_blk, col1_blk, col1_blk],
        out_specs=[row_blk, row_blk, col1_blk],
        out_shape=[
            jax.ShapeDtypeStruct((N, D), jnp.float32),
            jax.ShapeDtypeStruct((N, D), jnp.float32),
            jax.ShapeDtypeStruct((N, 1), jnp.float32),
        ],
    )(x, W1, Wsk, bsk[None, :], d0, d1)

    scatter = _make_scatter_kernel(A, D, n_chunks)
    s1 = scatter(hs1, src_p, dst_p)

    hs2 = pl.pallas_call(
        _mid_body,
        grid=(n_blk,),
        in_specs=[row_blk, row_blk, row_blk, row_blk, col1_blk,
                  vec_blk, vec_blk, vec_blk, w_blk],
        out_specs=row_blk,
        out_shape=jax.ShapeDtypeStruct((N, D), jnp.float32),
    )(s1[0, :N], s1[1, :N], hs1, skip, dinv,
      b1[None, :], g1[None, :], be1[None, :], W2)

    s2 = scatter(hs2, src_p, dst_p)

    ptr_f = ptr.astype(jnp.float32)
    lo_r = ptr_f[:-1][None, :]
    hi_r = ptr_f[1:][None, :]
    lo_c = ptr_f[:-1][:, None]
    hi_c = ptr_f[1:][:, None]

    h2, pooled = pl.pallas_call(
        functools.partial(_post_body, rows_per_blk=R, n_blk=n_blk),
        grid=(n_blk,),
        in_specs=[row_blk, row_blk, row_blk, col1_blk,
                  vec_blk, vec_blk, vec_blk,
                  pl.BlockSpec((1, NB), lambda i: (0, 0)),
                  pl.BlockSpec((1, NB), lambda i: (0, 0)),
                  pl.BlockSpec((NB, 1), lambda i: (0, 0)),
                  pl.BlockSpec((NB, 1), lambda i: (0, 0))],
        out_specs=[row_blk, pl.BlockSpec((NB, D), lambda i: (0, 0))],
        out_shape=[
            jax.ShapeDtypeStruct((N, D), jnp.float32),
            jax.ShapeDtypeStruct((NB, D), jnp.float32),
        ],
        scratch_shapes=[pltpu.VMEM((NB, D), jnp.float32)],
    )(s2[0, :N], s2[1, :N], hs2, dinv,
      b2[None, :], g2[None, :], be2[None, :], lo_r, hi_r, lo_c, hi_c)

    return (h2, pooled)


# speedup vs baseline: 29.3230x; 1.4870x over previous
import functools

import jax
import jax.numpy as jnp
from jax import lax
from jax.experimental import pallas as pl
from jax.experimental.pallas import tpu as pltpu
from jax.experimental.pallas import tpu_sc as plsc

NC = 2
NS = 16
NW = NC * NS
CHUNK = 64


def _sc_mesh():
    return plsc.VectorSubcoreMesh(core_axis_name="c", subcore_axis_name="s",
                                  num_cores=NC, num_subcores=NS)


def _make_deg_kernel(A, n_chunks):
    @functools.partial(
        pl.kernel,
        out_type=jax.ShapeDtypeStruct((NC, A), jnp.float32),
        mesh=_sc_mesh(),
        scratch_types=[
            pltpu.VMEM((n_chunks + 3, 2, CHUNK), jnp.int32),
            pltpu.VMEM((CHUNK,), jnp.float32),
            pltpu.VMEM((A // NS,), jnp.float32),
            pltpu.VMEM_SHARED((A,), jnp.float32),
            pltpu.SemaphoreType.DMA,
            pltpu.SemaphoreType.DMA,
        ],
    )
    def deg_kernel(idx_hbm, out_hbm, didx, ones, vbuf, dacc, sem0, sem1):
        c = lax.axis_index("c")
        s = lax.axis_index("s")
        w = c * NS + s
        per_tile = A // NS
        zeros16 = jnp.zeros((16,), jnp.float32)
        ones16 = jnp.ones((16,), jnp.float32)
        sems = (sem0, sem1)

        pltpu.sync_copy(idx_hbm.at[w], didx)

        def zb(i, _):
            vbuf[pl.ds(i * 16, 16)] = zeros16
            return 0

        lax.fori_loop(0, per_tile // 16, zb, 0)
        for j in range(CHUNK // 16):
            ones[pl.ds(j * 16, 16)] = ones16
        pltpu.sync_copy(vbuf, dacc.at[pl.ds(s * per_tile, per_tile)])
        plsc.subcore_barrier()

        def pair(p, _):
            for b in range(2):
                i = 2 * p + b
                o = 1 - b
                pltpu.async_copy(ones, dacc.at[didx.at[i, 1]], sems[b],
                                 add=True)

                @pl.when(i > 0)
                def _():
                    pltpu.make_async_copy(
                        ones, dacc.at[didx.at[i - 1, 1]], sems[o]).wait()
            return 0

        lax.fori_loop(0, n_chunks // 2, pair, 0)
        pltpu.make_async_copy(
            ones, dacc.at[didx.at[n_chunks - 1, 1]], sems[1]).wait()
        plsc.subcore_barrier()
        pltpu.sync_copy(dacc.at[pl.ds(s * per_tile, per_tile)], vbuf)
        pltpu.sync_copy(vbuf, out_hbm.at[c, pl.ds(s * per_tile, per_tile)])

    return deg_kernel


def _make_scatter_kernel(A, D, n_chunks):
    NBUF = 4

    @functools.partial(
        pl.kernel,
        out_type=jax.ShapeDtypeStruct((NC, A, D), jnp.float32),
        mesh=_sc_mesh(),
        scratch_types=(
            [pltpu.VMEM((2, CHUNK), jnp.int32) for _ in range(NBUF)]
            + [pltpu.VMEM((CHUNK, D), jnp.float32) for _ in range(NBUF)]
            + [pltpu.VMEM_SHARED((A, D), jnp.float32)]
            + [pltpu.SemaphoreType.DMA] * (3 * NBUF)
        ),
    )
    def scatter_kernel(h_hbm, idx_hbm, out_hbm, *bufs):
        idx = bufs[0:NBUF]
        rows = bufs[NBUF:2 * NBUF]
        acc = bufs[2 * NBUF]
        isem = bufs[2 * NBUF + 1:2 * NBUF + 1 + NBUF]
        gsem = bufs[2 * NBUF + 1 + NBUF:2 * NBUF + 1 + 2 * NBUF]
        ssem = bufs[2 * NBUF + 1 + 2 * NBUF:2 * NBUF + 1 + 3 * NBUF]
        c = lax.axis_index("c")
        s = lax.axis_index("s")
        w = c * NS + s
        per_tile = A // NS
        n_out = per_tile // CHUNK
        zeros16 = jnp.zeros((16,), jnp.float32)

        for b in range(NBUF - 1):
            pltpu.async_copy(idx_hbm.at[w, b], idx[b], isem[b])

        def zb(i, _):
            r = i // (D // 16)
            col = (i % (D // 16)) * 16
            rows[0][r, pl.ds(col, 16)] = zeros16
            return 0

        lax.fori_loop(0, CHUNK * (D // 16), zb, 0)
        for k in range(n_out):
            pltpu.sync_copy(rows[0],
                            acc.at[pl.ds(s * per_tile + k * CHUNK, CHUNK)])
        plsc.subcore_barrier()

        for b in range(2):
            pltpu.make_async_copy(idx_hbm.at[w, b], idx[b], isem[b]).wait()
            pltpu.async_copy(h_hbm.at[idx[b].at[0]], rows[b], gsem[b])

        def quad(p, _):
            for b in range(NBUF):
                i = NBUF * p + b
                nb = (b + NBUF - 1) % NBUF
                g2 = (b + 2) % NBUF
                pltpu.make_async_copy(h_hbm.at[idx[b].at[0]], rows[b],
                                      gsem[b]).wait()
                pltpu.async_copy(rows[b], acc.at[idx[b].at[1]], ssem[b],
                                 add=True)

                @pl.when(i > 0)
                def _():
                    pltpu.make_async_copy(rows[nb], acc.at[idx[nb].at[1]],
                                          ssem[nb]).wait()

                pltpu.async_copy(idx_hbm.at[w, i + 3], idx[nb], isem[nb])
                pltpu.make_async_copy(idx_hbm.at[w, i + 2], idx[g2],
                                      isem[g2]).wait()
                pltpu.async_copy(h_hbm.at[idx[g2].at[0]], rows[g2], gsem[g2])
            return 0

        lax.fori_loop(0, n_chunks // NBUF, quad, 0)
        pltpu.make_async_copy(rows[(n_chunks - 1) % NBUF],
                              acc.at[idx[(n_chunks - 1) % NBUF].at[1]],
                              ssem[(n_chunks - 1) % NBUF]).wait()
        for j in (n_chunks, n_chunks + 1):
            pltpu.make_async_copy(h_hbm.at[idx[j % NBUF].at[0]],
                                  rows[j % NBUF], gsem[j % NBUF]).wait()
        pltpu.make_async_copy(idx_hbm.at[w, n_chunks + 2],
                              idx[(n_chunks + 2) % NBUF],
                              isem[(n_chunks + 2) % NBUF]).wait()
        plsc.subcore_barrier()

        def oslice(k):
            return pl.ds(s * per_tile + k * CHUNK, CHUNK)

        pltpu.async_copy(acc.at[oslice(0)], rows[0], gsem[0])
        for k in range(n_out):
            b = k % 2
            o = 1 - b
            pltpu.make_async_copy(acc.at[oslice(k)], rows[b], gsem[b]).wait()
            pltpu.async_copy(rows[b], out_hbm.at[c, oslice(k)], ssem[b])
            if k + 1 < n_out:
                if k >= 1:
                    pltpu.make_async_copy(rows[o], out_hbm.at[c, oslice(k - 1)],
                                          ssem[o]).wait()
                pltpu.async_copy(acc.at[oslice(k + 1)], rows[o], gsem[o])
        if n_out >= 2:
            pltpu.make_async_copy(rows[(n_out - 2) % 2],
                                  out_hbm.at[c, oslice(n_out - 2)],
                                  ssem[(n_out - 2) % 2]).wait()
        pltpu.make_async_copy(rows[(n_out - 1) % 2],
                              out_hbm.at[c, oslice(n_out - 1)],
                              ssem[(n_out - 1) % 2]).wait()

    return scatter_kernel


def _pre_body(x_ref, w1_ref, wsk_ref, bsk_ref, d0_ref, d1_ref,
              hs1_ref, skip_ref, dinv_ref):
    deg = d0_ref[...] + d1_ref[...] + 1.0
    dinv = lax.rsqrt(deg)
    h1 = jnp.dot(x_ref[...], w1_ref[...], preferred_element_type=jnp.float32)
    hs1_ref[...] = h1 * dinv
    skip_ref[...] = (
        jnp.dot(x_ref[...], wsk_ref[...], preferred_element_type=jnp.float32)
        + bsk_ref[...])
    dinv_ref[...] = dinv


def _layer_norm(u, g, b):
    m = jnp.mean(u, axis=-1, keepdims=True)
    v = jnp.mean((u - m) ** 2, axis=-1, keepdims=True)
    return (u - m) * lax.rsqrt(v + 1e-5) * g + b


def _mid_body(p0_ref, p1_ref, hs1_ref, skip_ref, dinv_ref, b1_ref, g1_ref,
              be1_ref, w2_ref, hs2_ref):
    dinv = dinv_ref[...]
    agg = (p0_ref[...] + p1_ref[...] + hs1_ref[...]) * dinv + b1_ref[...]
    h = jax.nn.relu(_layer_norm(agg, g1_ref[...], be1_ref[...]))
    u = skip_ref[...] + h
    hs2_ref[...] = (
        jnp.dot(u, w2_ref[...], preferred_element_type=jnp.float32) * dinv)


def _post_body(q0_ref, q1_ref, hs2_ref, dinv_ref, b2_ref, g2_ref, be2_ref,
               lo_r_ref, hi_r_ref, lo_c_ref, hi_c_ref,
               h2_ref, pool_ref, pool_acc, *, rows_per_blk, n_blk):
    agg = ((q0_ref[...] + q1_ref[...] + hs2_ref[...]) * dinv_ref[...]
           + b2_ref[...])
    y = jax.nn.relu(_layer_norm(agg, g2_ref[...], be2_ref[...]))
    h2_ref[...] = y
    i = pl.program_id(0)
    ridx = (lax.broadcasted_iota(jnp.int32, (rows_per_blk, 1), 0)
            + rows_per_blk * i).astype(jnp.float32)
    onehot = ((ridx >= lo_r_ref[...]) & (ridx < hi_r_ref[...])
              ).astype(jnp.float32)
    part = lax.dot_general(onehot, y, (((0,), (0,)), ((), ())),
                           preferred_element_type=jnp.float32)

    @pl.when(i == 0)
    def _():
        pool_acc[...] = jnp.zeros_like(pool_acc)

    pool_acc[...] += part

    @pl.when(i == n_blk - 1)
    def _():
        cnt = jnp.maximum(hi_c_ref[...] - lo_c_ref[...], 1.0)
        pool_ref[...] = pool_acc[...] / cnt


def kernel(x, edge_index, ptr, W1, b1, g1, be1, W2, b2, g2, be2, Wsk, bsk):
    N, D = x.shape
    E = edge_index.shape[1]
    NB = ptr.shape[0] - 1

    PAD_ROWS = 240
    A = N + PAD_ROWS
    assert A % (NS * CHUNK) == 0

    src = edge_index[0].astype(jnp.int32)
    dst = edge_index[1].astype(jnp.int32)
    n_chunks = 4 * (-(-E // (NW * CHUNK * 4)))
    Ep = NW * n_chunks * CHUNK
    pad = Ep - E
    pad_pos = jnp.arange(pad, dtype=jnp.int32)
    src_p = jnp.concatenate([src, (pad_pos * 97) % N]).reshape(
        NW, n_chunks, CHUNK)
    dst_p = jnp.concatenate([dst, N + (pad_pos % PAD_ROWS)]).reshape(
        NW, n_chunks, CHUNK)
    xpos = jnp.arange(NW * 3 * CHUNK, dtype=jnp.int32)
    src_x = ((xpos * 131) % N).reshape(NW, 3, CHUNK)
    dst_x = (N + (xpos % PAD_ROWS)).reshape(NW, 3, CHUNK)
    src_p = jnp.concatenate([src_p, src_x], axis=1)
    dst_p = jnp.concatenate([dst_p, dst_x], axis=1)
    idx_p = jnp.stack([src_p, dst_p], axis=2)

    degp = _make_deg_kernel(A, n_chunks)(idx_p)
    d0 = degp[0, :N, None]
    d1 = degp[1, :N, None]

    R = 1000
    n_blk = N // R
    row_blk = pl.BlockSpec((R, D), lambda i: (i, 0))
    col1_blk = pl.BlockSpec((R, 1), lambda i: (i, 0))
    w_blk = pl.BlockSpec((D, D), lambda i: (0, 0))
    vec_blk = pl.BlockSpec((1, D), lambda i: (0, 0))

    hs1, skip, dinv = pl.pallas_call(
        _pre_body,
        grid=(n_blk,),
        in_specs=[row_blk, w_blk, w_blk, vec_blk, col1_blk, col1_blk],
        out_specs=[row_blk, row_blk, col1_blk],
        out_shape=[
            jax.ShapeDtypeStruct((N, D), jnp.float32),
            jax.ShapeDtypeStruct((N, D), jnp.float32),
            jax.ShapeDtypeStruct((N, 1), jnp.float32),
        ],
    )(x, W1, Wsk, bsk[None, :], d0, d1)

    scatter = _make_scatter_kernel(A, D, n_chunks)
    s1 = scatter(hs1, idx_p)

    hs2 = pl.pallas_call(
        _mid_body,
        grid=(n_blk,),
        in_specs=[row_blk, row_blk, row_blk, row_blk, col1_blk,
                  vec_blk, vec_blk, vec_blk, w_blk],
        out_specs=row_blk,
        out_shape=jax.ShapeDtypeStruct((N, D), jnp.float32),
    )(s1[0, :N], s1[1, :N], hs1, skip, dinv,
      b1[None, :], g1[None, :], be1[None, :], W2)

    s2 = scatter(hs2, idx_p)

    ptr_f = ptr.astype(jnp.float32)
    lo_r = ptr_f[:-1][None, :]
    hi_r = ptr_f[1:][None, :]
    lo_c = ptr_f[:-1][:, None]
    hi_c = ptr_f[1:][:, None]

    h2, pooled = pl.pallas_call(
        functools.partial(_post_body, rows_per_blk=R, n_blk=n_blk),
        grid=(n_blk,),
        in_specs=[row_blk, row_blk, row_blk, col1_blk,
                  vec_blk, vec_blk, vec_blk,
                  pl.BlockSpec((1, NB), lambda i: (0, 0)),
                  pl.BlockSpec((1, NB), lambda i: (0, 0)),
                  pl.BlockSpec((NB, 1), lambda i: (0, 0)),
                  pl.BlockSpec((NB, 1), lambda i: (0, 0))],
        out_specs=[row_blk, pl.BlockSpec((NB, D), lambda i: (0, 0))],
        out_shape=[
            jax.ShapeDtypeStruct((N, D), jnp.float32),
            jax.ShapeDtypeStruct((NB, D), jnp.float32),
        ],
        scratch_shapes=[pltpu.VMEM((NB, D), jnp.float32)],
    )(s2[0, :N], s2[1, :N], hs2, dinv,
      b2[None, :], g2[None, :], be2[None, :], lo_r, hi_r, lo_c, hi_c)

    return (h2, pooled)


# speedup vs baseline: 32.6244x; 1.1126x over previous
import functools

import jax
import jax.numpy as jnp
from jax import lax
from jax.experimental import pallas as pl
from jax.experimental.pallas import tpu as pltpu
from jax.experimental.pallas import tpu_sc as plsc

NC = 2
NS = 16
NW = NC * NS
CHUNK = 80


def _sc_mesh():
    return plsc.VectorSubcoreMesh(core_axis_name="c", subcore_axis_name="s",
                                  num_cores=NC, num_subcores=NS)


def _make_deg_kernel(A, n_chunks):
    @functools.partial(
        pl.kernel,
        out_type=jax.ShapeDtypeStruct((NC, A), jnp.float32),
        mesh=_sc_mesh(),
        scratch_types=[
            pltpu.VMEM((n_chunks + 3, 2, CHUNK), jnp.int32),
            pltpu.VMEM((CHUNK,), jnp.float32),
            pltpu.VMEM((A // NS,), jnp.float32),
            pltpu.VMEM_SHARED((A,), jnp.float32),
            pltpu.SemaphoreType.DMA,
            pltpu.SemaphoreType.DMA,
        ],
    )
    def deg_kernel(idx_hbm, out_hbm, didx, ones, vbuf, dacc, sem0, sem1):
        c = lax.axis_index("c")
        s = lax.axis_index("s")
        w = c * NS + s
        per_tile = A // NS
        zeros16 = jnp.zeros((16,), jnp.float32)
        ones16 = jnp.ones((16,), jnp.float32)
        sems = (sem0, sem1)

        pltpu.sync_copy(idx_hbm.at[w], didx)

        def zb(i, _):
            vbuf[pl.ds(i * 16, 16)] = zeros16
            return 0

        lax.fori_loop(0, per_tile // 16, zb, 0)
        for j in range(CHUNK // 16):
            ones[pl.ds(j * 16, 16)] = ones16
        pltpu.sync_copy(vbuf, dacc.at[pl.ds(s * per_tile, per_tile)])
        plsc.subcore_barrier()

        def pair(p, _):
            for b in range(2):
                i = 2 * p + b
                o = 1 - b
                pltpu.async_copy(ones, dacc.at[didx.at[i, 1]], sems[b],
                                 add=True)

                @pl.when(i > 0)
                def _():
                    pltpu.make_async_copy(
                        ones, dacc.at[didx.at[i - 1, 1]], sems[o]).wait()
            return 0

        lax.fori_loop(0, n_chunks // 2, pair, 0)
        pltpu.make_async_copy(
            ones, dacc.at[didx.at[n_chunks - 1, 1]], sems[1]).wait()
        plsc.subcore_barrier()
        pltpu.sync_copy(dacc.at[pl.ds(s * per_tile, per_tile)], vbuf)
        pltpu.sync_copy(vbuf, out_hbm.at[c, pl.ds(s * per_tile, per_tile)])

    return deg_kernel


def _make_scatter_kernel(A, D, n_chunks):
    NBUF = 4

    @functools.partial(
        pl.kernel,
        out_type=jax.ShapeDtypeStruct((NC, A, D), jnp.float32),
        mesh=_sc_mesh(),
        scratch_types=(
            [pltpu.VMEM((2, CHUNK), jnp.int32) for _ in range(NBUF)]
            + [pltpu.VMEM((CHUNK, D), jnp.float32) for _ in range(NBUF)]
            + [pltpu.VMEM_SHARED((A, D), jnp.float32)]
            + [pltpu.SemaphoreType.DMA] * (3 * NBUF)
        ),
    )
    def scatter_kernel(h_hbm, idx_hbm, out_hbm, *bufs):
        idx = bufs[0:NBUF]
        rows = bufs[NBUF:2 * NBUF]
        acc = bufs[2 * NBUF]
        isem = bufs[2 * NBUF + 1:2 * NBUF + 1 + NBUF]
        gsem = bufs[2 * NBUF + 1 + NBUF:2 * NBUF + 1 + 2 * NBUF]
        ssem = bufs[2 * NBUF + 1 + 2 * NBUF:2 * NBUF + 1 + 3 * NBUF]
        c = lax.axis_index("c")
        s = lax.axis_index("s")
        w = c * NS + s
        per_tile = A // NS
        n_out = per_tile // CHUNK
        zeros16 = jnp.zeros((16,), jnp.float32)

        for b in range(NBUF - 1):
            pltpu.async_copy(idx_hbm.at[w, b], idx[b], isem[b])

        def zb(i, _):
            r = i // (D // 16)
            col = (i % (D // 16)) * 16
            rows[0][r, pl.ds(col, 16)] = zeros16
            return 0

        lax.fori_loop(0, CHUNK * (D // 16), zb, 0)
        for k in range(n_out):
            pltpu.sync_copy(rows[0],
                            acc.at[pl.ds(s * per_tile + k * CHUNK, CHUNK)])
        plsc.subcore_barrier()

        for b in range(2):
            pltpu.make_async_copy(idx_hbm.at[w, b], idx[b], isem[b]).wait()
            pltpu.async_copy(h_hbm.at[idx[b].at[0]], rows[b], gsem[b])

        def quad(p, _):
            for b in range(NBUF):
                i = NBUF * p + b
                nb = (b + NBUF - 1) % NBUF
                g2 = (b + 2) % NBUF
                pltpu.make_async_copy(h_hbm.at[idx[b].at[0]], rows[b],
                                      gsem[b]).wait()
                pltpu.async_copy(rows[b], acc.at[idx[b].at[1]], ssem[b],
                                 add=True)

                @pl.when(i > 0)
                def _():
                    pltpu.make_async_copy(rows[nb], acc.at[idx[nb].at[1]],
                                          ssem[nb]).wait()

                pltpu.async_copy(idx_hbm.at[w, i + 3], idx[nb], isem[nb])
                pltpu.make_async_copy(idx_hbm.at[w, i + 2], idx[g2],
                                      isem[g2]).wait()
                pltpu.async_copy(h_hbm.at[idx[g2].at[0]], rows[g2], gsem[g2])
            return 0

        lax.fori_loop(0, n_chunks // NBUF, quad, 0)
        pltpu.make_async_copy(rows[(n_chunks - 1) % NBUF],
                              acc.at[idx[(n_chunks - 1) % NBUF].at[1]],
                              ssem[(n_chunks - 1) % NBUF]).wait()
        for j in (n_chunks, n_chunks + 1):
            pltpu.make_async_copy(h_hbm.at[idx[j % NBUF].at[0]],
                                  rows[j % NBUF], gsem[j % NBUF]).wait()
        pltpu.make_async_copy(idx_hbm.at[w, n_chunks + 2],
                              idx[(n_chunks + 2) % NBUF],
                              isem[(n_chunks + 2) % NBUF]).wait()
        plsc.subcore_barrier()

        def oslice(k):
            return pl.ds(s * per_tile + k * CHUNK, CHUNK)

        pltpu.async_copy(acc.at[oslice(0)], rows[0], gsem[0])
        for k in range(n_out):
            b = k % 2
            o = 1 - b
            pltpu.make_async_copy(acc.at[oslice(k)], rows[b], gsem[b]).wait()
            pltpu.async_copy(rows[b], out_hbm.at[c, oslice(k)], ssem[b])
            if k + 1 < n_out:
                if k >= 1:
                    pltpu.make_async_copy(rows[o], out_hbm.at[c, oslice(k - 1)],
                                          ssem[o]).wait()
                pltpu.async_copy(acc.at[oslice(k + 1)], rows[o], gsem[o])
        if n_out >= 2:
            pltpu.make_async_copy(rows[(n_out - 2) % 2],
                                  out_hbm.at[c, oslice(n_out - 2)],
                                  ssem[(n_out - 2) % 2]).wait()
        pltpu.make_async_copy(rows[(n_out - 1) % 2],
                              out_hbm.at[c, oslice(n_out - 1)],
                              ssem[(n_out - 1) % 2]).wait()

    return scatter_kernel


def _pre_body(x_ref, w1_ref, d0_ref, d1_ref, hs1_ref, dinv_ref):
    deg = d0_ref[...] + d1_ref[...] + 1.0
    dinv = lax.rsqrt(deg)
    h1 = jnp.dot(x_ref[...], w1_ref[...], preferred_element_type=jnp.float32)
    hs1_ref[...] = h1 * dinv
    dinv_ref[...] = dinv


def _layer_norm(u, g, b):
    m = jnp.mean(u, axis=-1, keepdims=True)
    v = jnp.mean((u - m) ** 2, axis=-1, keepdims=True)
    return (u - m) * lax.rsqrt(v + 1e-5) * g + b


def _mid_body(p0_ref, p1_ref, hs1_ref, x_ref, wsk_ref, bsk_ref, dinv_ref,
              b1_ref, g1_ref, be1_ref, w2_ref, hs2_ref):
    dinv = dinv_ref[...]
    agg = (p0_ref[...] + p1_ref[...] + hs1_ref[...]) * dinv + b1_ref[...]
    h = jax.nn.relu(_layer_norm(agg, g1_ref[...], be1_ref[...]))
    skip = (jnp.dot(x_ref[...], wsk_ref[...],
                    preferred_element_type=jnp.float32) + bsk_ref[...])
    u = skip + h
    hs2_ref[...] = (
        jnp.dot(u, w2_ref[...], preferred_element_type=jnp.float32) * dinv)


def _post_body(q0_ref, q1_ref, hs2_ref, dinv_ref, b2_ref, g2_ref, be2_ref,
               lo_r_ref, hi_r_ref, lo_c_ref, hi_c_ref,
               h2_ref, pool_ref, pool_acc, *, rows_per_blk, n_blk):
    agg = ((q0_ref[...] + q1_ref[...] + hs2_ref[...]) * dinv_ref[...]
           + b2_ref[...])
    y = jax.nn.relu(_layer_norm(agg, g2_ref[...], be2_ref[...]))
    h2_ref[...] = y
    i = pl.program_id(0)
    ridx = (lax.broadcasted_iota(jnp.int32, (rows_per_blk, 1), 0)
            + rows_per_blk * i).astype(jnp.float32)
    onehot = ((ridx >= lo_r_ref[...]) & (ridx < hi_r_ref[...])
              ).astype(jnp.float32)
    part = lax.dot_general(onehot, y, (((0,), (0,)), ((), ())),
                           preferred_element_type=jnp.float32)

    @pl.when(i == 0)
    def _():
        pool_acc[...] = jnp.zeros_like(pool_acc)

    pool_acc[...] += part

    @pl.when(i == n_blk - 1)
    def _():
        cnt = jnp.maximum(hi_c_ref[...] - lo_c_ref[...], 1.0)
        pool_ref[...] = pool_acc[...] / cnt


def kernel(x, edge_index, ptr, W1, b1, g1, be1, W2, b2, g2, be2, Wsk, bsk):
    N, D = x.shape
    E = edge_index.shape[1]
    NB = ptr.shape[0] - 1

    PAD_ROWS = 240
    A = N + PAD_ROWS
    assert A % (NS * CHUNK) == 0

    src = edge_index[0].astype(jnp.int32)
    dst = edge_index[1].astype(jnp.int32)
    n_chunks = 4 * (-(-E // (NW * CHUNK * 4)))
    Ep = NW * n_chunks * CHUNK
    pad = Ep - E
    pad_pos = jnp.arange(pad, dtype=jnp.int32)
    src_p = jnp.concatenate([src, (pad_pos * 97) % N]).reshape(
        NW, n_chunks, CHUNK)
    dst_p = jnp.concatenate([dst, N + (pad_pos % PAD_ROWS)]).reshape(
        NW, n_chunks, CHUNK)
    xpos = jnp.arange(NW * 3 * CHUNK, dtype=jnp.int32)
    src_x = ((xpos * 131) % N).reshape(NW, 3, CHUNK)
    dst_x = (N + (xpos % PAD_ROWS)).reshape(NW, 3, CHUNK)
    src_p = jnp.concatenate([src_p, src_x], axis=1)
    dst_p = jnp.concatenate([dst_p, dst_x], axis=1)
    idx_p = jnp.stack([src_p, dst_p], axis=2)

    degp = _make_deg_kernel(A, n_chunks)(idx_p)
    degp = degp[:, :, None]

    R = 1000
    n_blk = N // R
    row_blk = pl.BlockSpec((R, D), lambda i: (i, 0))
    col1_blk = pl.BlockSpec((R, 1), lambda i: (i, 0))
    w_blk = pl.BlockSpec((D, D), lambda i: (0, 0))
    vec_blk = pl.BlockSpec((1, D), lambda i: (0, 0))
    part0_blk = pl.BlockSpec((1, R, D), lambda i: (0, i, 0))
    part1_blk = pl.BlockSpec((1, R, D), lambda i: (1, i, 0))
    deg0_blk = pl.BlockSpec((1, R, 1), lambda i: (0, i, 0))
    deg1_blk = pl.BlockSpec((1, R, 1), lambda i: (1, i, 0))

    def _pre_wrap(x_ref, w1_ref, d0_ref, d1_ref, hs1_ref, dinv_ref):
        _pre_body(x_ref, w1_ref, d0_ref.at[0], d1_ref.at[0],
                  hs1_ref, dinv_ref)

    hs1, dinv = pl.pallas_call(
        _pre_wrap,
        grid=(n_blk,),
        in_specs=[row_blk, w_blk, deg0_blk, deg1_blk],
        out_specs=[row_blk, col1_blk],
        out_shape=[
            jax.ShapeDtypeStruct((N, D), jnp.float32),
            jax.ShapeDtypeStruct((N, 1), jnp.float32),
        ],
    )(x, W1, degp, degp)

    scatter = _make_scatter_kernel(A, D, n_chunks)
    s1 = scatter(hs1, idx_p)

    def _mid_wrap(p0_ref, p1_ref, hs1_ref, x_ref, wsk_ref, bsk_ref,
                  dinv_ref, b1_ref, g1_ref, be1_ref, w2_ref, hs2_ref):
        _mid_body(p0_ref.at[0], p1_ref.at[0], hs1_ref, x_ref, wsk_ref,
                  bsk_ref, dinv_ref, b1_ref, g1_ref, be1_ref, w2_ref,
                  hs2_ref)

    hs2 = pl.pallas_call(
        _mid_wrap,
        grid=(n_blk,),
        in_specs=[part0_blk, part1_blk, row_blk, row_blk, w_blk, vec_blk,
                  col1_blk, vec_blk, vec_blk, vec_blk, w_blk],
        out_specs=row_blk,
        out_shape=jax.ShapeDtypeStruct((N, D), jnp.float32),
    )(s1, s1, hs1, x, Wsk, bsk[None, :], dinv,
      b1[None, :], g1[None, :], be1[None, :], W2)

    s2 = scatter(hs2, idx_p)

    ptr_f = ptr.astype(jnp.float32)
    lo_r = ptr_f[:-1][None, :]
    hi_r = ptr_f[1:][None, :]
    lo_c = ptr_f[:-1][:, None]
    hi_c = ptr_f[1:][:, None]

    def _post_wrap(q0_ref, q1_ref, *rest):
        _post_body(q0_ref.at[0], q1_ref.at[0], *rest,
                   rows_per_blk=R, n_blk=n_blk)

    h2, pooled = pl.pallas_call(
        _post_wrap,
        grid=(n_blk,),
        in_specs=[part0_blk, part1_blk, row_blk, col1_blk,
                  vec_blk, vec_blk, vec_blk,
                  pl.BlockSpec((1, NB), lambda i: (0, 0)),
                  pl.BlockSpec((1, NB), lambda i: (0, 0)),
                  pl.BlockSpec((NB, 1), lambda i: (0, 0)),
                  pl.BlockSpec((NB, 1), lambda i: (0, 0))],
        out_specs=[row_blk, pl.BlockSpec((NB, D), lambda i: (0, 0))],
        out_shape=[
            jax.ShapeDtypeStruct((N, D), jnp.float32),
            jax.ShapeDtypeStruct((NB, D), jnp.float32),
        ],
        scratch_shapes=[pltpu.VMEM((NB, D), jnp.float32)],
    )(s2, s2, hs2, dinv,
      b2[None, :], g2[None, :], be2[None, :], lo_r, hi_r, lo_c, hi_c)

    return (h2, pooled)


# speedup vs baseline: 33.0255x; 1.0123x over previous
import functools

import jax
import jax.numpy as jnp
from jax import lax
from jax.experimental import pallas as pl
from jax.experimental.pallas import tpu as pltpu
from jax.experimental.pallas import tpu_sc as plsc

NC = 2
NS = 16
NW = NC * NS
CHUNK = 80


def _sc_mesh():
    return plsc.VectorSubcoreMesh(core_axis_name="c", subcore_axis_name="s",
                                  num_cores=NC, num_subcores=NS)


def _make_deg_kernel(A, n_chunks):
    @functools.partial(
        pl.kernel,
        out_type=jax.ShapeDtypeStruct((NC, A), jnp.float32),
        mesh=_sc_mesh(),
        scratch_types=[
            pltpu.VMEM((n_chunks + 3, 2, CHUNK), jnp.int32),
            pltpu.VMEM((CHUNK,), jnp.float32),
            pltpu.VMEM((A // NS,), jnp.float32),
            pltpu.VMEM_SHARED((A,), jnp.float32),
            pltpu.SemaphoreType.DMA,
            pltpu.SemaphoreType.DMA,
        ],
    )
    def deg_kernel(idx_hbm, out_hbm, didx, ones, vbuf, dacc, sem0, sem1):
        c = lax.axis_index("c")
        s = lax.axis_index("s")
        w = c * NS + s
        per_tile = A // NS
        zeros16 = jnp.zeros((16,), jnp.float32)
        ones16 = jnp.ones((16,), jnp.float32)
        sems = (sem0, sem1)

        pltpu.sync_copy(idx_hbm.at[w], didx)

        def zb(i, _):
            vbuf[pl.ds(i * 16, 16)] = zeros16
            return 0

        lax.fori_loop(0, per_tile // 16, zb, 0)
        for j in range(CHUNK // 16):
            ones[pl.ds(j * 16, 16)] = ones16
        pltpu.sync_copy(vbuf, dacc.at[pl.ds(s * per_tile, per_tile)])
        plsc.subcore_barrier()

        def pair(p, _):
            for b in range(2):
                i = 2 * p + b
                o = 1 - b
                pltpu.async_copy(ones, dacc.at[didx.at[i, 1]], sems[b],
                                 add=True)

                @pl.when(i > 0)
                def _():
                    pltpu.make_async_copy(
                        ones, dacc.at[didx.at[i - 1, 1]], sems[o]).wait()
            return 0

        lax.fori_loop(0, n_chunks // 2, pair, 0)
        pltpu.make_async_copy(
            ones, dacc.at[didx.at[n_chunks - 1, 1]], sems[1]).wait()
        plsc.subcore_barrier()
        pltpu.sync_copy(dacc.at[pl.ds(s * per_tile, per_tile)], vbuf)
        pltpu.sync_copy(vbuf, out_hbm.at[c, pl.ds(s * per_tile, per_tile)])

    return deg_kernel


def _make_scatter_kernel(A, D, n_chunks):
    NBUF = 4

    @functools.partial(
        pl.kernel,
        out_type=jax.ShapeDtypeStruct((NC, A, D), jnp.float32),
        mesh=_sc_mesh(),
        scratch_types=(
            [pltpu.VMEM((2, CHUNK), jnp.int32) for _ in range(NBUF)]
            + [pltpu.VMEM((CHUNK, D), jnp.float32) for _ in range(NBUF)]
            + [pltpu.VMEM_SHARED((A, D), jnp.float32)]
            + [pltpu.SemaphoreType.DMA] * (3 * NBUF)
        ),
    )
    def scatter_kernel(h_hbm, idx_hbm, out_hbm, *bufs):
        idx = bufs[0:NBUF]
        rows = bufs[NBUF:2 * NBUF]
        acc = bufs[2 * NBUF]
        isem = bufs[2 * NBUF + 1:2 * NBUF + 1 + NBUF]
        gsem = bufs[2 * NBUF + 1 + NBUF:2 * NBUF + 1 + 2 * NBUF]
        ssem = bufs[2 * NBUF + 1 + 2 * NBUF:2 * NBUF + 1 + 3 * NBUF]
        c = lax.axis_index("c")
        s = lax.axis_index("s")
        w = c * NS + s
        per_tile = A // NS
        n_out = per_tile // CHUNK
        zeros16 = jnp.zeros((16,), jnp.float32)

        for b in range(NBUF - 1):
            pltpu.async_copy(idx_hbm.at[w, b], idx[b], isem[b])

        def zb(i, _):
            r = i // (D // 16)
            col = (i % (D // 16)) * 16
            rows[3][r, pl.ds(col, 16)] = zeros16
            return 0

        lax.fori_loop(0, CHUNK * (D // 16), zb, 0)
        for k in range(n_out):
            pltpu.async_copy(rows[3],
                             acc.at[pl.ds(s * per_tile + k * CHUNK, CHUNK)],
                             ssem[3])
        for b in range(2):
            pltpu.make_async_copy(idx_hbm.at[w, b], idx[b], isem[b]).wait()
            pltpu.async_copy(h_hbm.at[idx[b].at[0]], rows[b], gsem[b])
        for k in range(n_out):
            pltpu.make_async_copy(
                rows[3], acc.at[pl.ds(s * per_tile + k * CHUNK, CHUNK)],
                ssem[3]).wait()
        plsc.subcore_barrier()

        def quad(p, _):
            for b in range(NBUF):
                i = NBUF * p + b
                nb = (b + NBUF - 1) % NBUF
                g2 = (b + 2) % NBUF
                pltpu.make_async_copy(h_hbm.at[idx[b].at[0]], rows[b],
                                      gsem[b]).wait()
                pltpu.async_copy(rows[b], acc.at[idx[b].at[1]], ssem[b],
                                 add=True)

                @pl.when(i > 0)
                def _():
                    pltpu.make_async_copy(rows[nb], acc.at[idx[nb].at[1]],
                                          ssem[nb]).wait()

                pltpu.async_copy(idx_hbm.at[w, i + 3], idx[nb], isem[nb])
                pltpu.make_async_copy(idx_hbm.at[w, i + 2], idx[g2],
                                      isem[g2]).wait()
                pltpu.async_copy(h_hbm.at[idx[g2].at[0]], rows[g2], gsem[g2])
            return 0

        lax.fori_loop(0, n_chunks // NBUF, quad, 0)
        pltpu.make_async_copy(rows[(n_chunks - 1) % NBUF],
                              acc.at[idx[(n_chunks - 1) % NBUF].at[1]],
                              ssem[(n_chunks - 1) % NBUF]).wait()
        for j in (n_chunks, n_chunks + 1):
            pltpu.make_async_copy(h_hbm.at[idx[j % NBUF].at[0]],
                                  rows[j % NBUF], gsem[j % NBUF]).wait()
        pltpu.make_async_copy(idx_hbm.at[w, n_chunks + 2],
                              idx[(n_chunks + 2) % NBUF],
                              isem[(n_chunks + 2) % NBUF]).wait()
        plsc.subcore_barrier()

        def oslice(k):
            return pl.ds(s * per_tile + k * CHUNK, CHUNK)

        pltpu.async_copy(acc.at[oslice(0)], rows[0], gsem[0])
        for k in range(n_out):
            b = k % 2
            o = 1 - b
            pltpu.make_async_copy(acc.at[oslice(k)], rows[b], gsem[b]).wait()
            pltpu.async_copy(rows[b], out_hbm.at[c, oslice(k)], ssem[b])
            if k + 1 < n_out:
                if k >= 1:
                    pltpu.make_async_copy(rows[o], out_hbm.at[c, oslice(k - 1)],
                                          ssem[o]).wait()
                pltpu.async_copy(acc.at[oslice(k + 1)], rows[o], gsem[o])
        if n_out >= 2:
            pltpu.make_async_copy(rows[(n_out - 2) % 2],
                                  out_hbm.at[c, oslice(n_out - 2)],
                                  ssem[(n_out - 2) % 2]).wait()
        pltpu.make_async_copy(rows[(n_out - 1) % 2],
                              out_hbm.at[c, oslice(n_out - 1)],
                              ssem[(n_out - 1) % 2]).wait()

    return scatter_kernel


def _pre_body(x_ref, w1_ref, d0_ref, d1_ref, hs1_ref, dinv_ref):
    deg = d0_ref[...] + d1_ref[...] + 1.0
    dinv = lax.rsqrt(deg)
    h1 = jnp.dot(x_ref[...], w1_ref[...], preferred_element_type=jnp.float32)
    hs1_ref[...] = h1 * dinv
    dinv_ref[...] = dinv


def _layer_norm(u, g, b):
    m = jnp.mean(u, axis=-1, keepdims=True)
    v = jnp.mean((u - m) ** 2, axis=-1, keepdims=True)
    return (u - m) * lax.rsqrt(v + 1e-5) * g + b


def _mid_body(p0_ref, p1_ref, hs1_ref, x_ref, wsk_ref, bsk_ref, dinv_ref,
              b1_ref, g1_ref, be1_ref, w2_ref, hs2_ref):
    dinv = dinv_ref[...]
    agg = (p0_ref[...] + p1_ref[...] + hs1_ref[...]) * dinv + b1_ref[...]
    h = jax.nn.relu(_layer_norm(agg, g1_ref[...], be1_ref[...]))
    skip = (jnp.dot(x_ref[...], wsk_ref[...],
                    preferred_element_type=jnp.float32) + bsk_ref[...])
    u = skip + h
    hs2_ref[...] = (
        jnp.dot(u, w2_ref[...], preferred_element_type=jnp.float32) * dinv)


def _post_body(q0_ref, q1_ref, hs2_ref, dinv_ref, b2_ref, g2_ref, be2_ref,
               lo_r_ref, hi_r_ref, lo_c_ref, hi_c_ref,
               h2_ref, pool_ref, pool_acc, *, rows_per_blk, n_blk):
    agg = ((q0_ref[...] + q1_ref[...] + hs2_ref[...]) * dinv_ref[...]
           + b2_ref[...])
    y = jax.nn.relu(_layer_norm(agg, g2_ref[...], be2_ref[...]))
    h2_ref[...] = y
    i = pl.program_id(0)
    ridx = (lax.broadcasted_iota(jnp.int32, (rows_per_blk, 1), 0)
            + rows_per_blk * i).astype(jnp.float32)
    onehot = ((ridx >= lo_r_ref[...]) & (ridx < hi_r_ref[...])
              ).astype(jnp.float32)
    part = lax.dot_general(onehot, y, (((0,), (0,)), ((), ())),
                           preferred_element_type=jnp.float32)

    @pl.when(i == 0)
    def _():
        pool_acc[...] = jnp.zeros_like(pool_acc)

    pool_acc[...] += part

    @pl.when(i == n_blk - 1)
    def _():
        cnt = jnp.maximum(hi_c_ref[...] - lo_c_ref[...], 1.0)
        pool_ref[...] = pool_acc[...] / cnt


def kernel(x, edge_index, ptr, W1, b1, g1, be1, W2, b2, g2, be2, Wsk, bsk):
    N, D = x.shape
    E = edge_index.shape[1]
    NB = ptr.shape[0] - 1

    PAD_ROWS = 240
    A = N + PAD_ROWS
    assert A % (NS * CHUNK) == 0

    src = edge_index[0].astype(jnp.int32)
    dst = edge_index[1].astype(jnp.int32)
    n_chunks = 4 * (-(-E // (NW * CHUNK * 4)))
    Ep = NW * n_chunks * CHUNK
    pad = Ep - E
    pad_pos = jnp.arange(pad, dtype=jnp.int32)
    src_p = jnp.concatenate([src, (pad_pos * 97) % N]).reshape(
        NW, n_chunks, CHUNK)
    dst_p = jnp.concatenate([dst, N + (pad_pos % PAD_ROWS)]).reshape(
        NW, n_chunks, CHUNK)
    xpos = jnp.arange(NW * 3 * CHUNK, dtype=jnp.int32)
    src_x = ((xpos * 131) % N).reshape(NW, 3, CHUNK)
    dst_x = (N + (xpos % PAD_ROWS)).reshape(NW, 3, CHUNK)
    src_p = jnp.concatenate([src_p, src_x], axis=1)
    dst_p = jnp.concatenate([dst_p, dst_x], axis=1)
    idx_p = jnp.stack([src_p, dst_p], axis=2)

    degp = _make_deg_kernel(A, n_chunks)(idx_p)
    degp = degp[:, :, None]

    R = 1000
    n_blk = N // R
    row_blk = pl.BlockSpec((R, D), lambda i: (i, 0))
    col1_blk = pl.BlockSpec((R, 1), lambda i: (i, 0))
    w_blk = pl.BlockSpec((D, D), lambda i: (0, 0))
    vec_blk = pl.BlockSpec((1, D), lambda i: (0, 0))
    part0_blk = pl.BlockSpec((1, R, D), lambda i: (0, i, 0))
    part1_blk = pl.BlockSpec((1, R, D), lambda i: (1, i, 0))
    deg0_blk = pl.BlockSpec((1, R, 1), lambda i: (0, i, 0))
    deg1_blk = pl.BlockSpec((1, R, 1), lambda i: (1, i, 0))

    def _pre_wrap(x_ref, w1_ref, d0_ref, d1_ref, hs1_ref, dinv_ref):
        _pre_body(x_ref, w1_ref, d0_ref.at[0], d1_ref.at[0],
                  hs1_ref, dinv_ref)

    hs1, dinv = pl.pallas_call(
        _pre_wrap,
        grid=(n_blk,),
        in_specs=[row_blk, w_blk, deg0_blk, deg1_blk],
        out_specs=[row_blk, col1_blk],
        out_shape=[
            jax.ShapeDtypeStruct((N, D), jnp.float32),
            jax.ShapeDtypeStruct((N, 1), jnp.float32),
        ],
    )(x, W1, degp, degp)

    scatter = _make_scatter_kernel(A, D, n_chunks)
    s1 = scatter(hs1, idx_p)

    def _mid_wrap(p0_ref, p1_ref, hs1_ref, x_ref, wsk_ref, bsk_ref,
                  dinv_ref, b1_ref, g1_ref, be1_ref, w2_ref, hs2_ref):
        _mid_body(p0_ref.at[0], p1_ref.at[0], hs1_ref, x_ref, wsk_ref,
                  bsk_ref, dinv_ref, b1_ref, g1_ref, be1_ref, w2_ref,
                  hs2_ref)

    hs2 = pl.pallas_call(
        _mid_wrap,
        grid=(n_blk,),
        in_specs=[part0_blk, part1_blk, row_blk, row_blk, w_blk, vec_blk,
                  col1_blk, vec_blk, vec_blk, vec_blk, w_blk],
        out_specs=row_blk,
        out_shape=jax.ShapeDtypeStruct((N, D), jnp.float32),
    )(s1, s1, hs1, x, Wsk, bsk[None, :], dinv,
      b1[None, :], g1[None, :], be1[None, :], W2)

    s2 = scatter(hs2, idx_p)

    ptr_f = ptr.astype(jnp.float32)
    lo_r = ptr_f[:-1][None, :]
    hi_r = ptr_f[1:][None, :]
    lo_c = ptr_f[:-1][:, None]
    hi_c = ptr_f[1:][:, None]

    def _post_wrap(q0_ref, q1_ref, *rest):
        _post_body(q0_ref.at[0], q1_ref.at[0], *rest,
                   rows_per_blk=R, n_blk=n_blk)

    h2, pooled = pl.pallas_call(
        _post_wrap,
        grid=(n_blk,),
        in_specs=[part0_blk, part1_blk, row_blk, col1_blk,
                  vec_blk, vec_blk, vec_blk,
                  pl.BlockSpec((1, NB), lambda i: (0, 0)),
                  pl.BlockSpec((1, NB), lambda i: (0, 0)),
                  pl.BlockSpec((NB, 1), lambda i: (0, 0)),
                  pl.BlockSpec((NB, 1), lambda i: (0, 0))],
        out_specs=[row_blk, pl.BlockSpec((NB, D), lambda i: (0, 0))],
        out_shape=[
            jax.ShapeDtypeStruct((N, D), jnp.float32),
            jax.ShapeDtypeStruct((NB, D), jnp.float32),
        ],
        scratch_shapes=[pltpu.VMEM((NB, D), jnp.float32)],
    )(s2, s2, hs2, dinv,
      b2[None, :], g2[None, :], be2[None, :], lo_r, hi_r, lo_c, hi_c)

    return (h2, pooled)


# speedup vs baseline: 35.7712x; 1.0831x over previous
import functools

import jax
import jax.numpy as jnp
from jax import lax
from jax.experimental import pallas as pl
from jax.experimental.pallas import tpu as pltpu
from jax.experimental.pallas import tpu_sc as plsc

NC = 2
NS = 16
NW = NC * NS
CHUNK = 80


def _sc_mesh():
    return plsc.VectorSubcoreMesh(core_axis_name="c", subcore_axis_name="s",
                                  num_cores=NC, num_subcores=NS)


def _make_deg_kernel(A, n_chunks):
    @functools.partial(
        pl.kernel,
        out_type=jax.ShapeDtypeStruct((NC, A), jnp.float32),
        mesh=_sc_mesh(),
        scratch_types=[
            pltpu.VMEM((n_chunks, 2, CHUNK), jnp.int32),
            pltpu.VMEM((CHUNK,), jnp.float32),
            pltpu.VMEM((A // NS,), jnp.float32),
            pltpu.VMEM_SHARED((A,), jnp.float32),
            pltpu.SemaphoreType.DMA,
            pltpu.SemaphoreType.DMA,
        ],
    )
    def deg_kernel(idx_hbm, out_hbm, didx, ones, vbuf, dacc, sem0, sem1):
        c = lax.axis_index("c")
        s = lax.axis_index("s")
        w = c * NS + s
        per_tile = A // NS
        zeros16 = jnp.zeros((16,), jnp.float32)
        ones16 = jnp.ones((16,), jnp.float32)
        sems = (sem0, sem1)

        pltpu.sync_copy(idx_hbm.at[w], didx)
        n_tail = n_chunks % 2

        def zb(i, _):
            vbuf[pl.ds(i * 16, 16)] = zeros16
            return 0

        lax.fori_loop(0, per_tile // 16, zb, 0)
        for j in range(CHUNK // 16):
            ones[pl.ds(j * 16, 16)] = ones16
        pltpu.sync_copy(vbuf, dacc.at[pl.ds(s * per_tile, per_tile)])
        plsc.subcore_barrier()

        def pair(p, _):
            for b in range(2):
                i = 2 * p + b
                o = 1 - b
                pltpu.async_copy(ones, dacc.at[didx.at[i, 1]], sems[b],
                                 add=True)

                @pl.when(i > 0)
                def _():
                    pltpu.make_async_copy(
                        ones, dacc.at[didx.at[i - 1, 1]], sems[o]).wait()
            return 0

        lax.fori_loop(0, n_chunks // 2, pair, 0)
        if n_tail:
            i = n_chunks - 1
            pltpu.async_copy(ones, dacc.at[didx.at[i, 1]], sems[i % 2],
                             add=True)
            pltpu.make_async_copy(
                ones, dacc.at[didx.at[i - 1, 1]], sems[(i - 1) % 2]).wait()
        pltpu.make_async_copy(
            ones, dacc.at[didx.at[n_chunks - 1, 1]],
            sems[(n_chunks - 1) % 2]).wait()
        plsc.subcore_barrier()
        pltpu.sync_copy(dacc.at[pl.ds(s * per_tile, per_tile)], vbuf)
        pltpu.sync_copy(vbuf, out_hbm.at[c, pl.ds(s * per_tile, per_tile)])

    return deg_kernel


def _make_scatter_kernel(A, D, n_chunks):
    NBUF = 4

    @functools.partial(
        pl.kernel,
        out_type=jax.ShapeDtypeStruct((NC, A, D), jnp.float32),
        mesh=_sc_mesh(),
        scratch_types=(
            [pltpu.VMEM((2, CHUNK), jnp.int32) for _ in range(NBUF)]
            + [pltpu.VMEM((CHUNK, D), jnp.float32) for _ in range(NBUF)]
            + [pltpu.VMEM_SHARED((A, D), jnp.float32)]
            + [pltpu.SemaphoreType.DMA] * (3 * NBUF)
        ),
    )
    def scatter_kernel(h_hbm, idx_hbm, out_hbm, *bufs):
        idx = bufs[0:NBUF]
        rows = bufs[NBUF:2 * NBUF]
        acc = bufs[2 * NBUF]
        isem = bufs[2 * NBUF + 1:2 * NBUF + 1 + NBUF]
        gsem = bufs[2 * NBUF + 1 + NBUF:2 * NBUF + 1 + 2 * NBUF]
        ssem = bufs[2 * NBUF + 1 + 2 * NBUF:2 * NBUF + 1 + 3 * NBUF]
        c = lax.axis_index("c")
        s = lax.axis_index("s")
        w = c * NS + s
        per_tile = A // NS
        n_out = per_tile // CHUNK
        zeros16 = jnp.zeros((16,), jnp.float32)

        for b in range(NBUF - 1):
            pltpu.async_copy(idx_hbm.at[w, b], idx[b], isem[b])

        def zb(i, _):
            r = i // (D // 16)
            col = (i % (D // 16)) * 16
            rows[3][r, pl.ds(col, 16)] = zeros16
            return 0

        lax.fori_loop(0, CHUNK * (D // 16), zb, 0)
        for k in range(n_out):
            pltpu.async_copy(rows[3],
                             acc.at[pl.ds(s * per_tile + k * CHUNK, CHUNK)],
                             ssem[3])
        for b in range(2):
            pltpu.make_async_copy(idx_hbm.at[w, b], idx[b], isem[b]).wait()
            pltpu.async_copy(h_hbm.at[idx[b].at[0]], rows[b], gsem[b])
        for k in range(n_out):
            pltpu.make_async_copy(
                rows[3], acc.at[pl.ds(s * per_tile + k * CHUNK, CHUNK)],
                ssem[3]).wait()
        plsc.subcore_barrier()

        def body(i, b, skip_swait):
            nb = (b + NBUF - 1) % NBUF
            g2 = (b + 2) % NBUF
            pltpu.make_async_copy(h_hbm.at[idx[b].at[0]], rows[b],
                                  gsem[b]).wait()
            pltpu.async_copy(rows[b], acc.at[idx[b].at[1]], ssem[b],
                             add=True)

            def swait():
                pltpu.make_async_copy(rows[nb], acc.at[idx[nb].at[1]],
                                      ssem[nb]).wait()

            if skip_swait:
                pl.when(i > 0)(swait)
            else:
                swait()
            pc = jnp.minimum(i + 3, n_chunks - 1)
            pltpu.async_copy(idx_hbm.at[w, pc], idx[nb], isem[nb])
            pltpu.make_async_copy(idx_hbm.at[w, 0], idx[g2],
                                  isem[g2]).wait()
            pltpu.async_copy(h_hbm.at[idx[g2].at[0]], rows[g2], gsem[g2])

        def quad(p, _):
            for b in range(NBUF):
                body(NBUF * p + b, b, skip_swait=(b == 0))
            return 0

        nq, n_tail = divmod(n_chunks, NBUF)
        lax.fori_loop(0, nq, quad, 0)
        for t in range(n_tail):
            body(NBUF * nq + t, t, skip_swait=False)
        pltpu.make_async_copy(rows[(n_chunks - 1) % NBUF],
                              acc.at[idx[(n_chunks - 1) % NBUF].at[1]],
                              ssem[(n_chunks - 1) % NBUF]).wait()
        for j in (n_chunks, n_chunks + 1):
            pltpu.make_async_copy(h_hbm.at[idx[j % NBUF].at[0]],
                                  rows[j % NBUF], gsem[j % NBUF]).wait()
        pltpu.make_async_copy(idx_hbm.at[w, 0],
                              idx[(n_chunks + 2) % NBUF],
                              isem[(n_chunks + 2) % NBUF]).wait()
        plsc.subcore_barrier()

        def oslice(k):
            return pl.ds(s * per_tile + k * CHUNK, CHUNK)

        pltpu.async_copy(acc.at[oslice(0)], rows[0], gsem[0])
        for k in range(n_out):
            b = k % 2
            o = 1 - b
            pltpu.make_async_copy(acc.at[oslice(k)], rows[b], gsem[b]).wait()
            pltpu.async_copy(rows[b], out_hbm.at[c, oslice(k)], ssem[b])
            if k + 1 < n_out:
                if k >= 1:
                    pltpu.make_async_copy(rows[o], out_hbm.at[c, oslice(k - 1)],
                                          ssem[o]).wait()
                pltpu.async_copy(acc.at[oslice(k + 1)], rows[o], gsem[o])
        if n_out >= 2:
            pltpu.make_async_copy(rows[(n_out - 2) % 2],
                                  out_hbm.at[c, oslice(n_out - 2)],
                                  ssem[(n_out - 2) % 2]).wait()
        pltpu.make_async_copy(rows[(n_out - 1) % 2],
                              out_hbm.at[c, oslice(n_out - 1)],
                              ssem[(n_out - 1) % 2]).wait()

    return scatter_kernel


def _pre_body(x_ref, w1_ref, d0_ref, d1_ref, hs1_ref, dinv_ref):
    deg = d0_ref[...] + d1_ref[...] + 1.0
    dinv = lax.rsqrt(deg)
    h1 = jnp.dot(x_ref[...], w1_ref[...], preferred_element_type=jnp.float32)
    hs1_ref[...] = h1 * dinv
    dinv_ref[...] = dinv


def _layer_norm(u, g, b):
    m = jnp.mean(u, axis=-1, keepdims=True)
    v = jnp.mean((u - m) ** 2, axis=-1, keepdims=True)
    return (u - m) * lax.rsqrt(v + 1e-5) * g + b


def _mid_body(p0_ref, p1_ref, hs1_ref, x_ref, wsk_ref, bsk_ref, dinv_ref,
              b1_ref, g1_ref, be1_ref, w2_ref, hs2_ref):
    dinv = dinv_ref[...]
    agg = (p0_ref[...] + p1_ref[...] + hs1_ref[...]) * dinv + b1_ref[...]
    h = jax.nn.relu(_layer_norm(agg, g1_ref[...], be1_ref[...]))
    skip = (jnp.dot(x_ref[...], wsk_ref[...],
                    preferred_element_type=jnp.float32) + bsk_ref[...])
    u = skip + h
    hs2_ref[...] = (
        jnp.dot(u, w2_ref[...], preferred_element_type=jnp.float32) * dinv)


def _post_body(q0_ref, q1_ref, hs2_ref, dinv_ref, b2_ref, g2_ref, be2_ref,
               lo_r_ref, hi_r_ref, lo_c_ref, hi_c_ref,
               h2_ref, pool_ref, pool_acc, *, rows_per_blk, n_blk):
    agg = ((q0_ref[...] + q1_ref[...] + hs2_ref[...]) * dinv_ref[...]
           + b2_ref[...])
    y = jax.nn.relu(_layer_norm(agg, g2_ref[...], be2_ref[...]))
    h2_ref[...] = y
    i = pl.program_id(0)
    ridx = (lax.broadcasted_iota(jnp.int32, (rows_per_blk, 1), 0)
            + rows_per_blk * i).astype(jnp.float32)
    onehot = ((ridx >= lo_r_ref[...]) & (ridx < hi_r_ref[...])
              ).astype(jnp.float32)
    part = lax.dot_general(onehot, y, (((0,), (0,)), ((), ())),
                           preferred_element_type=jnp.float32)

    @pl.when(i == 0)
    def _():
        pool_acc[...] = jnp.zeros_like(pool_acc)

    pool_acc[...] += part

    @pl.when(i == n_blk - 1)
    def _():
        cnt = jnp.maximum(hi_c_ref[...] - lo_c_ref[...], 1.0)
        pool_ref[...] = pool_acc[...] / cnt


def kernel(x, edge_index, ptr, W1, b1, g1, be1, W2, b2, g2, be2, Wsk, bsk):
    N, D = x.shape
    E = edge_index.shape[1]
    NB = ptr.shape[0] - 1

    PAD_ROWS = 240
    A = N + PAD_ROWS
    assert A % (NS * CHUNK) == 0

    assert E % (NW * CHUNK) == 0
    n_chunks = E // (NW * CHUNK)
    eidx = edge_index.astype(jnp.int32).reshape(2, NW, n_chunks, CHUNK)
    idx_p = eidx.transpose(1, 2, 0, 3)

    degp = _make_deg_kernel(A, n_chunks)(idx_p)
    degp = degp[:, :, None]

    R = 1000
    n_blk = N // R
    row_blk = pl.BlockSpec((R, D), lambda i: (i, 0))
    col1_blk = pl.BlockSpec((R, 1), lambda i: (i, 0))
    w_blk = pl.BlockSpec((D, D), lambda i: (0, 0))
    vec_blk = pl.BlockSpec((1, D), lambda i: (0, 0))
    part0_blk = pl.BlockSpec((1, R, D), lambda i: (0, i, 0))
    part1_blk = pl.BlockSpec((1, R, D), lambda i: (1, i, 0))
    deg0_blk = pl.BlockSpec((1, R, 1), lambda i: (0, i, 0))
    deg1_blk = pl.BlockSpec((1, R, 1), lambda i: (1, i, 0))

    def _pre_wrap(x_ref, w1_ref, d0_ref, d1_ref, hs1_ref, dinv_ref):
        _pre_body(x_ref, w1_ref, d0_ref.at[0], d1_ref.at[0],
                  hs1_ref, dinv_ref)

    hs1, dinv = pl.pallas_call(
        _pre_wrap,
        grid=(n_blk,),
        in_specs=[row_blk, w_blk, deg0_blk, deg1_blk],
        out_specs=[row_blk, col1_blk],
        out_shape=[
            jax.ShapeDtypeStruct((N, D), jnp.float32),
            jax.ShapeDtypeStruct((N, 1), jnp.float32),
        ],
    )(x, W1, degp, degp)

    scatter = _make_scatter_kernel(A, D, n_chunks)
    s1 = scatter(hs1, idx_p)

    def _mid_wrap(p0_ref, p1_ref, hs1_ref, x_ref, wsk_ref, bsk_ref,
                  dinv_ref, b1_ref, g1_ref, be1_ref, w2_ref, hs2_ref):
        _mid_body(p0_ref.at[0], p1_ref.at[0], hs1_ref, x_ref, wsk_ref,
                  bsk_ref, dinv_ref, b1_ref, g1_ref, be1_ref, w2_ref,
                  hs2_ref)

    hs2 = pl.pallas_call(
        _mid_wrap,
        grid=(n_blk,),
        in_specs=[part0_blk, part1_blk, row_blk, row_blk, w_blk, vec_blk,
                  col1_blk, vec_blk, vec_blk, vec_blk, w_blk],
        out_specs=row_blk,
        out_shape=jax.ShapeDtypeStruct((N, D), jnp.float32),
    )(s1, s1, hs1, x, Wsk, bsk[None, :], dinv,
      b1[None, :], g1[None, :], be1[None, :], W2)

    s2 = scatter(hs2, idx_p)

    ptr_f = ptr.astype(jnp.float32)
    lo_r = ptr_f[:-1][None, :]
    hi_r = ptr_f[1:][None, :]
    lo_c = ptr_f[:-1][:, None]
    hi_c = ptr_f[1:][:, None]

    def _post_wrap(q0_ref, q1_ref, *rest):
        _post_body(q0_ref.at[0], q1_ref.at[0], *rest,
                   rows_per_blk=R, n_blk=n_blk)

    h2, pooled = pl.pallas_call(
        _post_wrap,
        grid=(n_blk,),
        in_specs=[part0_blk, part1_blk, row_blk, col1_blk,
                  vec_blk, vec_blk, vec_blk,
                  pl.BlockSpec((1, NB), lambda i: (0, 0)),
                  pl.BlockSpec((1, NB), lambda i: (0, 0)),
                  pl.BlockSpec((NB, 1), lambda i: (0, 0)),
                  pl.BlockSpec((NB, 1), lambda i: (0, 0))],
        out_specs=[row_blk, pl.BlockSpec((NB, D), lambda i: (0, 0))],
        out_shape=[
            jax.ShapeDtypeStruct((N, D), jnp.float32),
            jax.ShapeDtypeStruct((NB, D), jnp.float32),
        ],
        scratch_shapes=[pltpu.VMEM((NB, D), jnp.float32)],
    )(s2, s2, hs2, dinv,
      b2[None, :], g2[None, :], be2[None, :], lo_r, hi_r, lo_c, hi_c)

    return (h2, pooled)


# speedup vs baseline: 36.7909x; 1.0285x over previous
import functools

import jax
import jax.numpy as jnp
from jax import lax
from jax.experimental import pallas as pl
from jax.experimental.pallas import tpu as pltpu
from jax.experimental.pallas import tpu_sc as plsc

NC = 2
NS = 16
NW = NC * NS
CHUNK = 80


def _sc_mesh():
    return plsc.VectorSubcoreMesh(core_axis_name="c", subcore_axis_name="s",
                                  num_cores=NC, num_subcores=NS)


def _make_deg_kernel(A, n_chunks):
    @functools.partial(
        pl.kernel,
        out_type=jax.ShapeDtypeStruct((NC, A), jnp.float32),
        mesh=_sc_mesh(),
        scratch_types=[
            pltpu.VMEM((n_chunks, 2, CHUNK), jnp.int32),
            pltpu.VMEM((CHUNK,), jnp.float32),
            pltpu.VMEM((A // NS,), jnp.float32),
            pltpu.VMEM_SHARED((A,), jnp.float32),
            pltpu.SemaphoreType.DMA,
            pltpu.SemaphoreType.DMA,
        ],
    )
    def deg_kernel(idx_hbm, out_hbm, didx, ones, vbuf, dacc, sem0, sem1):
        c = lax.axis_index("c")
        s = lax.axis_index("s")
        w = c * NS + s
        per_tile = A // NS
        zeros16 = jnp.zeros((16,), jnp.float32)
        ones16 = jnp.ones((16,), jnp.float32)
        sems = (sem0, sem1)

        pltpu.sync_copy(idx_hbm.at[w], didx)
        n_tail = n_chunks % 2

        def zb(i, _):
            vbuf[pl.ds(i * 16, 16)] = zeros16
            return 0

        lax.fori_loop(0, per_tile // 16, zb, 0)
        for j in range(CHUNK // 16):
            ones[pl.ds(j * 16, 16)] = ones16
        pltpu.sync_copy(vbuf, dacc.at[pl.ds(s * per_tile, per_tile)])
        plsc.subcore_barrier()

        def pair(p, _):
            for b in range(2):
                i = 2 * p + b
                o = 1 - b
                pltpu.async_copy(ones, dacc.at[didx.at[i, 1]], sems[b],
                                 add=True)

                @pl.when(i > 0)
                def _():
                    pltpu.make_async_copy(
                        ones, dacc.at[didx.at[i - 1, 1]], sems[o]).wait()
            return 0

        lax.fori_loop(0, n_chunks // 2, pair, 0)
        if n_tail:
            i = n_chunks - 1
            pltpu.async_copy(ones, dacc.at[didx.at[i, 1]], sems[i % 2],
                             add=True)
            pltpu.make_async_copy(
                ones, dacc.at[didx.at[i - 1, 1]], sems[(i - 1) % 2]).wait()
        pltpu.make_async_copy(
            ones, dacc.at[didx.at[n_chunks - 1, 1]],
            sems[(n_chunks - 1) % 2]).wait()
        plsc.subcore_barrier()
        pltpu.sync_copy(dacc.at[pl.ds(s * per_tile, per_tile)], vbuf)
        pltpu.sync_copy(vbuf, out_hbm.at[c, pl.ds(s * per_tile, per_tile)])

    return deg_kernel


def _make_scatter_kernel(A, D, n_chunks):
    NBUF = 4

    @functools.partial(
        pl.kernel,
        out_type=jax.ShapeDtypeStruct((NC, A, D), jnp.float32),
        mesh=_sc_mesh(),
        scratch_types=(
            [pltpu.VMEM((2, CHUNK), jnp.int32) for _ in range(NBUF)]
            + [pltpu.VMEM((CHUNK, D), jnp.float32) for _ in range(NBUF)]
            + [pltpu.VMEM_SHARED((A, D), jnp.float32)]
            + [pltpu.SemaphoreType.DMA] * (3 * NBUF)
        ),
    )
    def scatter_kernel(h_hbm, idx_hbm, out_hbm, *bufs):
        idx = bufs[0:NBUF]
        rows = bufs[NBUF:2 * NBUF]
        acc = bufs[2 * NBUF]
        isem = bufs[2 * NBUF + 1:2 * NBUF + 1 + NBUF]
        gsem = bufs[2 * NBUF + 1 + NBUF:2 * NBUF + 1 + 2 * NBUF]
        ssem = bufs[2 * NBUF + 1 + 2 * NBUF:2 * NBUF + 1 + 3 * NBUF]
        c = lax.axis_index("c")
        s = lax.axis_index("s")
        w = c * NS + s
        per_tile = A // NS
        n_out = per_tile // CHUNK
        zeros16 = jnp.zeros((16,), jnp.float32)

        for b in range(NBUF - 1):
            pltpu.async_copy(idx_hbm.at[w, b], idx[b], isem[b])

        def zb(i, _):
            r = i // (D // 16)
            col = (i % (D // 16)) * 16
            rows[3][r, pl.ds(col, 16)] = zeros16
            return 0

        lax.fori_loop(0, CHUNK * (D // 16), zb, 0)
        for k in range(n_out):
            pltpu.async_copy(rows[3],
                             acc.at[pl.ds(s * per_tile + k * CHUNK, CHUNK)],
                             ssem[3])
        for b in range(2):
            pltpu.make_async_copy(idx_hbm.at[w, b], idx[b], isem[b]).wait()
            pltpu.async_copy(h_hbm.at[idx[b].at[0]], rows[b], gsem[b])
        for k in range(n_out):
            pltpu.make_async_copy(
                rows[3], acc.at[pl.ds(s * per_tile + k * CHUNK, CHUNK)],
                ssem[3]).wait()
        plsc.subcore_barrier()

        def body(i, b, skip_swait):
            nb = (b + NBUF - 1) % NBUF
            g2 = (b + 2) % NBUF
            pltpu.make_async_copy(h_hbm.at[idx[b].at[0]], rows[b],
                                  gsem[b]).wait()
            pltpu.async_copy(rows[b], acc.at[idx[b].at[1]], ssem[b],
                             add=True)

            def swait():
                pltpu.make_async_copy(rows[nb], acc.at[idx[nb].at[1]],
                                      ssem[nb]).wait()

            if skip_swait:
                pl.when(i > 0)(swait)
            else:
                swait()
            pc = jnp.minimum(i + 3, n_chunks - 1)
            pltpu.async_copy(idx_hbm.at[w, pc], idx[nb], isem[nb])
            pltpu.make_async_copy(idx_hbm.at[w, 0], idx[g2],
                                  isem[g2]).wait()
            pltpu.async_copy(h_hbm.at[idx[g2].at[0]], rows[g2], gsem[g2])

        def quad(p, _):
            for b in range(NBUF):
                body(NBUF * p + b, b, skip_swait=(b == 0))
            return 0

        nq, n_tail = divmod(n_chunks, NBUF)
        lax.fori_loop(0, nq, quad, 0)
        for t in range(n_tail):
            body(NBUF * nq + t, t, skip_swait=False)
        pltpu.make_async_copy(rows[(n_chunks - 1) % NBUF],
                              acc.at[idx[(n_chunks - 1) % NBUF].at[1]],
                              ssem[(n_chunks - 1) % NBUF]).wait()
        for j in (n_chunks, n_chunks + 1):
            pltpu.make_async_copy(h_hbm.at[idx[j % NBUF].at[0]],
                                  rows[j % NBUF], gsem[j % NBUF]).wait()
        pltpu.make_async_copy(idx_hbm.at[w, 0],
                              idx[(n_chunks + 2) % NBUF],
                              isem[(n_chunks + 2) % NBUF]).wait()
        plsc.subcore_barrier()

        def oslice(k):
            return pl.ds(s * per_tile + k * CHUNK, CHUNK)

        pltpu.async_copy(acc.at[oslice(0)], rows[0], gsem[0])
        for k in range(n_out):
            b = k % 2
            o = 1 - b
            pltpu.make_async_copy(acc.at[oslice(k)], rows[b], gsem[b]).wait()
            pltpu.async_copy(rows[b], out_hbm.at[c, oslice(k)], ssem[b])
            if k + 1 < n_out:
                if k >= 1:
                    pltpu.make_async_copy(rows[o], out_hbm.at[c, oslice(k - 1)],
                                          ssem[o]).wait()
                pltpu.async_copy(acc.at[oslice(k + 1)], rows[o], gsem[o])
        if n_out >= 2:
            pltpu.make_async_copy(rows[(n_out - 2) % 2],
                                  out_hbm.at[c, oslice(n_out - 2)],
                                  ssem[(n_out - 2) % 2]).wait()
        pltpu.make_async_copy(rows[(n_out - 1) % 2],
                              out_hbm.at[c, oslice(n_out - 1)],
                              ssem[(n_out - 1) % 2]).wait()

    return scatter_kernel


def _pre_body(x_ref, w1_ref, d0_ref, d1_ref, hs1_ref, dinv_ref):
    deg = d0_ref[...] + d1_ref[...] + 1.0
    dinv = lax.rsqrt(deg)
    h1 = jnp.dot(x_ref[...], w1_ref[...], preferred_element_type=jnp.float32)
    hs1_ref[...] = h1 * dinv
    dinv_ref[...] = dinv


def _layer_norm(u, g, b):
    m = jnp.mean(u, axis=-1, keepdims=True)
    v = jnp.mean((u - m) ** 2, axis=-1, keepdims=True)
    return (u - m) * lax.rsqrt(v + 1e-5) * g + b


def _mid_body(p0_ref, p1_ref, hs1_ref, x_ref, wsk_ref, bsk_ref, dinv_ref,
              b1_ref, g1_ref, be1_ref, w2_ref, hs2_ref):
    dinv = dinv_ref[...]
    agg = (p0_ref[...] + p1_ref[...] + hs1_ref[...]) * dinv + b1_ref[...]
    h = jax.nn.relu(_layer_norm(agg, g1_ref[...], be1_ref[...]))
    skip = (jnp.dot(x_ref[...], wsk_ref[...],
                    preferred_element_type=jnp.float32) + bsk_ref[...])
    u = skip + h
    hs2_ref[...] = (
        jnp.dot(u, w2_ref[...], preferred_element_type=jnp.float32) * dinv)


def _post_body(q0_ref, q1_ref, hs2_ref, dinv_ref, b2_ref, g2_ref, be2_ref,
               lo_r_ref, hi_r_ref, lo_c_ref, hi_c_ref,
               h2_ref, pool_ref, pool_acc, *, rows_per_blk, n_blk):
    agg = ((q0_ref[...] + q1_ref[...] + hs2_ref[...]) * dinv_ref[...]
           + b2_ref[...])
    y = jax.nn.relu(_layer_norm(agg, g2_ref[...], be2_ref[...]))
    h2_ref[...] = y
    i = pl.program_id(0)
    ridx = (lax.broadcasted_iota(jnp.int32, (rows_per_blk, 1), 0)
            + rows_per_blk * i).astype(jnp.float32)
    onehot = ((ridx >= lo_r_ref[...]) & (ridx < hi_r_ref[...])
              ).astype(jnp.float32)
    part = lax.dot_general(onehot, y, (((0,), (0,)), ((), ())),
                           preferred_element_type=jnp.float32)

    @pl.when(i == 0)
    def _():
        pool_acc[...] = jnp.zeros_like(pool_acc)

    pool_acc[...] += part

    @pl.when(i == n_blk - 1)
    def _():
        cnt = jnp.maximum(hi_c_ref[...] - lo_c_ref[...], 1.0)
        pool_ref[...] = pool_acc[...] / cnt


def kernel(x, edge_index, ptr, W1, b1, g1, be1, W2, b2, g2, be2, Wsk, bsk):
    N, D = x.shape
    E = edge_index.shape[1]
    NB = ptr.shape[0] - 1

    PAD_ROWS = 240
    A = N + PAD_ROWS
    assert A % (NS * CHUNK) == 0

    assert E % (NW * CHUNK) == 0
    n_chunks = E // (NW * CHUNK)
    eidx = edge_index.astype(jnp.int32).reshape(2, NW, n_chunks, CHUNK)
    idx_p = eidx.transpose(1, 2, 0, 3)

    degp = _make_deg_kernel(A, n_chunks)(idx_p)
    degp = degp[:, :, None]

    R = 2000
    n_blk = N // R
    row_blk = pl.BlockSpec((R, D), lambda i: (i, 0))
    col1_blk = pl.BlockSpec((R, 1), lambda i: (i, 0))
    w_blk = pl.BlockSpec((D, D), lambda i: (0, 0))
    vec_blk = pl.BlockSpec((1, D), lambda i: (0, 0))
    part0_blk = pl.BlockSpec((1, R, D), lambda i: (0, i, 0))
    part1_blk = pl.BlockSpec((1, R, D), lambda i: (1, i, 0))
    deg0_blk = pl.BlockSpec((1, R, 1), lambda i: (0, i, 0))
    deg1_blk = pl.BlockSpec((1, R, 1), lambda i: (1, i, 0))

    def _pre_wrap(x_ref, w1_ref, d0_ref, d1_ref, hs1_ref, dinv_ref):
        _pre_body(x_ref, w1_ref, d0_ref.at[0], d1_ref.at[0],
                  hs1_ref, dinv_ref)

    hs1, dinv = pl.pallas_call(
        _pre_wrap,
        grid=(n_blk,),
        in_specs=[row_blk, w_blk, deg0_blk, deg1_blk],
        out_specs=[row_blk, col1_blk],
        out_shape=[
            jax.ShapeDtypeStruct((N, D), jnp.float32),
            jax.ShapeDtypeStruct((N, 1), jnp.float32),
        ],
    )(x, W1, degp, degp)

    scatter = _make_scatter_kernel(A, D, n_chunks)
    s1 = scatter(hs1, idx_p)

    def _mid_wrap(p0_ref, p1_ref, hs1_ref, x_ref, wsk_ref, bsk_ref,
                  dinv_ref, b1_ref, g1_ref, be1_ref, w2_ref, hs2_ref):
        _mid_body(p0_ref.at[0], p1_ref.at[0], hs1_ref, x_ref, wsk_ref,
                  bsk_ref, dinv_ref, b1_ref, g1_ref, be1_ref, w2_ref,
                  hs2_ref)

    hs2 = pl.pallas_call(
        _mid_wrap,
        grid=(n_blk,),
        in_specs=[part0_blk, part1_blk, row_blk, row_blk, w_blk, vec_blk,
                  col1_blk, vec_blk, vec_blk, vec_blk, w_blk],
        out_specs=row_blk,
        out_shape=jax.ShapeDtypeStruct((N, D), jnp.float32),
    )(s1, s1, hs1, x, Wsk, bsk[None, :], dinv,
      b1[None, :], g1[None, :], be1[None, :], W2)

    s2 = scatter(hs2, idx_p)

    ptr_f = ptr.astype(jnp.float32)
    lo_r = ptr_f[:-1][None, :]
    hi_r = ptr_f[1:][None, :]
    lo_c = ptr_f[:-1][:, None]
    hi_c = ptr_f[1:][:, None]

    def _post_wrap(q0_ref, q1_ref, *rest):
        _post_body(q0_ref.at[0], q1_ref.at[0], *rest,
                   rows_per_blk=R, n_blk=n_blk)

    h2, pooled = pl.pallas_call(
        _post_wrap,
        grid=(n_blk,),
        in_specs=[part0_blk, part1_blk, row_blk, col1_blk,
                  vec_blk, vec_blk, vec_blk,
                  pl.BlockSpec((1, NB), lambda i: (0, 0)),
                  pl.BlockSpec((1, NB), lambda i: (0, 0)),
                  pl.BlockSpec((NB, 1), lambda i: (0, 0)),
                  pl.BlockSpec((NB, 1), lambda i: (0, 0))],
        out_specs=[row_blk, pl.BlockSpec((NB, D), lambda i: (0, 0))],
        out_shape=[
            jax.ShapeDtypeStruct((N, D), jnp.float32),
            jax.ShapeDtypeStruct((NB, D), jnp.float32),
        ],
        scratch_shapes=[pltpu.VMEM((NB, D), jnp.float32)],
    )(s2, s2, hs2, dinv,
      b2[None, :], g2[None, :], be2[None, :], lo_r, hi_r, lo_c, hi_c)

    return (h2, pooled)
